# Initial kernel scaffold; baseline (speedup 1.0000x reference)
import jax
import jax.numpy as jnp
from jax.experimental import pallas as pl


def kernel(input, node_embedding, edge_attr, edge_index, u, batch, params):
    raise NotImplementedError("write your pallas kernel here")


# speedup vs baseline: 5.7972x; 5.7972x over previous
import functools

import jax
import jax.numpy as jnp
from jax import lax
from jax.experimental import pallas as pl
from jax.experimental.pallas import tpu as pltpu
from jax.experimental.pallas import tpu_sc as plsc

F32 = jnp.float32
_PHI = jax.lax.Precision.DEFAULT
_PHX = jax.lax.Precision.HIGHEST
_INTERP = False


_NC, _NS, _L = 2, 16, 16
_NW = _NC * _NS
_GRP = 8
_SUB = 4


def _sc_mesh():
    return plsc.VectorSubcoreMesh(core_axis_name="c", subcore_axis_name="s")


def _sc_gather_rows(table, idx2d, width, out_rows):
    R = idx2d.shape[0]
    per_w = R // _NW
    n_steps = per_w // _GRP
    cs = _SUB * 128

    @functools.partial(
        pl.kernel,
        out_type=jax.ShapeDtypeStruct((out_rows, width), F32),
        mesh=_sc_mesh(),
        scratch_types=[
            pltpu.VMEM((_GRP, 128), jnp.int32),
            pltpu.VMEM((cs, width), F32),
            pltpu.SemaphoreType.DMA,
        ],
    )
    def k(table_hbm, idx_hbm, out_hbm, idx_v, dat_v, sem):
        w = lax.axis_index("c") * _NS + lax.axis_index("s")
        row0 = w * per_w

        def body(t, _):
            r0 = row0 + t * _GRP
            pltpu.sync_copy(idx_hbm.at[pl.ds(r0, _GRP)], idx_v)
            for half in range(_GRP // _SUB):
                for j in range(_SUB):
                    pltpu.async_copy(
                        table_hbm.at[idx_v.at[half * _SUB + j]],
                        dat_v.at[pl.ds(j * 128, 128)], sem).wait()
                pltpu.sync_copy(
                    dat_v, out_hbm.at[pl.ds((r0 + half * _SUB) * 128, cs)])
            return 0

        lax.fori_loop(0, n_steps, body, 0)

    return k(table, idx2d)


def _sc_gather_elems(table, idx2d, out_rows):
    R = idx2d.shape[0]
    per_w = R // _NW
    n_steps = per_w // _GRP
    cs = _GRP * 128

    @functools.partial(
        pl.kernel,
        out_type=jax.ShapeDtypeStruct((out_rows,), jnp.int32),
        mesh=_sc_mesh(),
        scratch_types=[
            pltpu.VMEM((_GRP, 128), jnp.int32),
            pltpu.VMEM((cs,), jnp.int32),
            pltpu.SemaphoreType.DMA,
        ],
    )
    def k(table_hbm, idx_hbm, out_hbm, idx_v, dat_v, sem):
        w = lax.axis_index("c") * _NS + lax.axis_index("s")
        row0 = w * per_w

        def body(t, _):
            r0 = row0 + t * _GRP
            pltpu.sync_copy(idx_hbm.at[pl.ds(r0, _GRP)], idx_v)
            for j in range(_GRP):
                pltpu.async_copy(
                    table_hbm.at[idx_v.at[j]],
                    dat_v.at[pl.ds(j * 128, 128)], sem).wait()
            pltpu.sync_copy(dat_v, out_hbm.at[pl.ds(r0 * 128, cs)])
            return 0

        lax.fori_loop(0, n_steps, body, 0)

    return k(table, idx2d)


def _sc_scatter_add(data, idx3d, n_rows, zeros_tab):
    epad, width = data.shape
    R = idx3d.shape[1]
    half_n = n_rows // 2
    tab_rows = half_n + 8
    per_s = R // _NS
    n_steps = per_s // _GRP
    cs = _SUB * 128
    zmain, ztail = (tab_rows // _NS) // 8 * 8, None
    ztail = tab_rows - zmain * _NS
    emain = (half_n // _NS) // 8 * 8
    etail = half_n - emain * _NS

    @functools.partial(
        pl.kernel,
        out_type=jax.ShapeDtypeStruct((n_rows, width), F32),
        mesh=_sc_mesh(),
        scratch_types=[
            pltpu.VMEM((_GRP, 128), jnp.int32),
            pltpu.VMEM((cs, width), F32),
            pltpu.VMEM_SHARED((tab_rows, width), F32),
        ],
    )
    def k(data_hbm, idx_hbm, zeros_hbm, out_hbm, idx_v, dat_v, shared):
        c = lax.axis_index("c")
        s = lax.axis_index("s")
        pltpu.sync_copy(zeros_hbm.at[pl.ds(s * zmain, zmain)],
                        shared.at[pl.ds(s * zmain, zmain)])
        if ztail:
            @pl.when(s == _NS - 1)
            def _():
                pltpu.sync_copy(zeros_hbm.at[pl.ds(_NS * zmain, ztail)],
                                shared.at[pl.ds(_NS * zmain, ztail)])
        plsc.subcore_barrier()
        row0 = s * per_s

        def body(t, _):
            r0 = row0 + t * _GRP
            pltpu.sync_copy(idx_hbm.at[c, pl.ds(r0, _GRP)], idx_v)
            for h in range(_GRP // _SUB):
                pltpu.sync_copy(
                    data_hbm.at[pl.ds((r0 + h * _SUB) * 128, cs)], dat_v)
                for j in range(_SUB):
                    pltpu.sync_copy(dat_v.at[pl.ds(j * 128, 128)],
                                    shared.at[idx_v.at[h * _SUB + j]],
                                    add=True)
            return 0

        lax.fori_loop(0, n_steps, body, 0)
        plsc.subcore_barrier()
        pltpu.sync_copy(shared.at[pl.ds(s * emain, emain)],
                        out_hbm.at[pl.ds(c * half_n + s * emain, emain)])
        if etail:
            @pl.when(s == _NS - 1)
            def _():
                pltpu.sync_copy(
                    shared.at[pl.ds(_NS * emain, etail)],
                    out_hbm.at[pl.ds(c * half_n + _NS * emain, etail)])

    return k(data, idx3d, zeros_tab)


def _ln(x, g, b):
    m = jnp.mean(x, axis=-1, keepdims=True)
    d = x - m
    v = jnp.mean(d * d, axis=-1, keepdims=True)
    return d / jnp.sqrt(v + 1e-5) * g + b


def _node_prep_call(inp, emb, wi, bi, we, be, wf, bf, g1, b1,
                    wqkvs, bqkvs, bn):
    n = inp.shape[0]
    grid = (n // bn,)

    def body(inp_r, emb_r, wi_r, bi_r, we_r, be_r, wf_r, bf_r,
             g1_r, b1_r, wq_r, bq_r, x0_o, q_o, k_o, v_o, s_o):
        a = jnp.dot(inp_r[...], wi_r[...], preferred_element_type=F32, precision=_PHI) + bi_r[...]
        ne = jnp.dot(emb_r[...], we_r[...], preferred_element_type=F32, precision=_PHI) + be_r[...]
        comb = jnp.concatenate([ne, a, ne], axis=-1)
        x0 = jnp.dot(comb, wf_r[...], preferred_element_type=F32,
                     precision=_PHI) + bf_r[...]
        xn = _ln(x0, g1_r[...], b1_r[...])
        qkvs = jnp.dot(xn, wq_r[...], preferred_element_type=F32, precision=_PHI) + bq_r[...]
        x0_o[...] = x0
        q_o[...] = qkvs[:, 0:128]
        k_o[...] = qkvs[:, 128:256]
        v_o[...] = qkvs[:, 256:384]
        s_o[...] = qkvs[:, 384:512]

    row = lambda i: (i, 0)
    full = lambda i: (0, 0)
    oshape = jax.ShapeDtypeStruct((n, 128), F32)
    return pl.pallas_call(
        body,
        grid=grid,
        in_specs=[
            pl.BlockSpec((bn, 128), row), pl.BlockSpec((bn, 128), row),
            pl.BlockSpec((128, 128), full), pl.BlockSpec((1, 128), full),
            pl.BlockSpec((128, 128), full), pl.BlockSpec((1, 128), full),
            pl.BlockSpec((384, 128), full),
            pl.BlockSpec((1, 128), full), pl.BlockSpec((1, 128), full),
            pl.BlockSpec((1, 128), full),
            pl.BlockSpec((128, 512), full), pl.BlockSpec((1, 512), full),
        ],
        out_specs=[pl.BlockSpec((bn, 128), row)] * 5,
        out_shape=[oshape] * 5,
        interpret=_INTERP,
    )(inp, emb, wi, bi, we, be, wf, bf, g1, b1, wqkvs, bqkvs)


def _attn_edge_call(kg, vg, qg, ea, watt, batt, wenc, benc, sel, rep,
                    n_edges, be):
    epad = kg.shape[0]
    grid = (epad // be,)

    def body(kg_r, vg_r, qg_r, ea_r, watt_r, batt_r, wenc_r, benc_r,
             sel_r, rep_r, contrib_o, dencnt_o, e0_o):
        i = pl.program_id(0)
        rows = lax.broadcasted_iota(jnp.int32, (be, 1), 0) + i * be
        mask = (rows < n_edges).astype(F32)
        ep = jnp.dot(ea_r[...], watt_r[...], preferred_element_type=F32, precision=_PHI) + batt_r[...]
        ke = kg_r[...] + ep
        ve = vg_r[...] + ep
        alpha = jnp.dot(qg_r[...] * ke, sel_r[...],
                        preferred_element_type=F32, precision=_PHX) * 0.25
        ex = jnp.exp(alpha) * mask
        contrib_o[...] = ve * jnp.dot(ex, rep_r[...],
                                      preferred_element_type=F32,
                                      precision=_PHX)
        dencnt_o[...] = jnp.concatenate(
            [ex, jnp.broadcast_to(mask, (be, 8)), jnp.zeros((be, 112), F32)],
            axis=-1)
        e0_o[...] = (jnp.dot(ea_r[...], wenc_r[...],
                             preferred_element_type=F32, precision=_PHI) + benc_r[...]) * mask

    row = lambda i: (i, 0)
    full = lambda i: (0, 0)
    return pl.pallas_call(
        body,
        grid=grid,
        in_specs=[
            pl.BlockSpec((be, 128), row), pl.BlockSpec((be, 128), row),
            pl.BlockSpec((be, 128), row), pl.BlockSpec((be, 16), row),
            pl.BlockSpec((16, 128), full), pl.BlockSpec((1, 128), full),
            pl.BlockSpec((16, 128), full), pl.BlockSpec((1, 128), full),
            pl.BlockSpec((128, 8), full), pl.BlockSpec((8, 128), full),
        ],
        out_specs=[pl.BlockSpec((be, 128), row), pl.BlockSpec((be, 128), row),
                   pl.BlockSpec((be, 128), row)],
        out_shape=[jax.ShapeDtypeStruct((epad, 128), F32),
                   jax.ShapeDtypeStruct((epad, 128), F32),
                   jax.ShapeDtypeStruct((epad, 128), F32)],
        interpret=_INTERP,
    )(kg, vg, qg, ea, watt, batt, wenc, benc, sel, rep)


def _attn_node_call(x0, skip, msg, dencnt, rep, g2, b2, wf1, bf1,
                    wf2, bf2, fng, fnb, bn):
    n = x0.shape[0]
    grid = (n // bn,)

    def body(x0_r, sk_r, m_r, d_r, rep_r, g2_r, b2_r,
             wf1_r, bf1_r, wf2_r, bf2_r, fng_r, fnb_r, x_o):
        den = jnp.dot(d_r[:, 0:8], rep_r[...],
                      preferred_element_type=F32, precision=_PHX) + 1e-16
        x1 = x0_r[...] + m_r[...] / den + sk_r[...]
        xn = _ln(x1, g2_r[...], b2_r[...])
        h = jnp.maximum(
            jnp.dot(xn, wf1_r[...], preferred_element_type=F32, precision=_PHI) + bf1_r[...], 0.0)
        ff = jnp.dot(h, wf2_r[...], preferred_element_type=F32, precision=_PHI) + bf2_r[...]
        x_o[...] = _ln(x1 + ff, fng_r[...], fnb_r[...])

    row = lambda i: (i, 0)
    full = lambda i: (0, 0)
    return pl.pallas_call(
        body,
        grid=grid,
        in_specs=[
            pl.BlockSpec((bn, 128), row), pl.BlockSpec((bn, 128), row),
            pl.BlockSpec((bn, 128), row), pl.BlockSpec((bn, 128), row),
            pl.BlockSpec((8, 128), full),
            pl.BlockSpec((1, 128), full), pl.BlockSpec((1, 128), full),
            pl.BlockSpec((128, 512), full), pl.BlockSpec((1, 512), full),
            pl.BlockSpec((512, 128), full), pl.BlockSpec((1, 128), full),
            pl.BlockSpec((1, 128), full), pl.BlockSpec((1, 128), full),
        ],
        out_specs=[pl.BlockSpec((bn, 128), row)],
        out_shape=[jax.ShapeDtypeStruct((n, 128), F32)],
        interpret=_INTERP,
    )(x0, skip, msg, dencnt, rep, g2, b2, wf1, bf1, wf2, bf2, fng, fnb)[0]


def _glob_prep_call(u, wg, bg, w1ue, b1e, w1un, b1n):
    def body(u_r, wg_r, bg_r, wue_r, be_r, wun_r, bn_r, uh_o, uue_o, uun_o):
        uh = jnp.dot(u_r[...], wg_r[...], preferred_element_type=F32,
                     precision=_PHI) + bg_r[...]
        uh_o[...] = uh
        uue_o[...] = jnp.dot(uh, wue_r[...], preferred_element_type=F32, precision=_PHI) + be_r[...]
        uun_o[...] = jnp.dot(uh, wun_r[...], preferred_element_type=F32, precision=_PHI) + bn_r[...]

    full = lambda: (0, 0)
    return pl.pallas_call(
        body,
        grid=(1,),
        in_specs=[
            pl.BlockSpec((16, 16), lambda i: (0, 0)),
            pl.BlockSpec((16, 128), lambda i: (0, 0)),
            pl.BlockSpec((1, 128), lambda i: (0, 0)),
            pl.BlockSpec((128, 512), lambda i: (0, 0)),
            pl.BlockSpec((1, 512), lambda i: (0, 0)),
            pl.BlockSpec((128, 512), lambda i: (0, 0)),
            pl.BlockSpec((1, 512), lambda i: (0, 0)),
        ],
        out_specs=[pl.BlockSpec((16, 128), lambda i: (0, 0)),
                   pl.BlockSpec((16, 512), lambda i: (0, 0)),
                   pl.BlockSpec((16, 512), lambda i: (0, 0))],
        out_shape=[jax.ShapeDtypeStruct((16, 128), F32),
                   jax.ShapeDtypeStruct((16, 512), F32),
                   jax.ShapeDtypeStruct((16, 512), F32)],
        interpret=_INTERP,
    )(u, wg, bg, w1ue, b1e, w1un, b1n)


def _edge_mlp_call(xg, xcg, e, br, w1r, w1c, w1e, uue, w2, b2, n_edges, be):
    epad = xg.shape[0]
    grid = (epad // be,)

    def body(xg_r, xcg_r, e_r, br_r, w1r_r, w1c_r, w1e_r, uue_r, w2_r, b2_r,
             enew_o, em_o, ce_o):
        i = pl.program_id(0)
        rows = lax.broadcasted_iota(jnp.int32, (be, 1), 0) + i * be
        mask = (rows < n_edges).astype(F32)
        oh = (br_r[...] == lax.broadcasted_iota(jnp.int32, (be, 16), 1)
              ).astype(F32) * mask
        pre = (jnp.dot(xg_r[...], w1r_r[...], preferred_element_type=F32, precision=_PHI)
               + jnp.dot(xcg_r[...], w1c_r[...], preferred_element_type=F32, precision=_PHI)
               + jnp.dot(e_r[...], w1e_r[...], preferred_element_type=F32, precision=_PHI)
               + jnp.dot(oh, uue_r[...], preferred_element_type=F32,
                         precision=_PHX))
        act = jnp.maximum(pre, 0.0)
        en = (jnp.dot(act, w2_r[...], preferred_element_type=F32, precision=_PHI)
              + b2_r[...]) * mask
        enew_o[...] = en

        @pl.when(i == 0)
        def _():
            em_o[...] = jnp.zeros_like(em_o)
            ce_o[...] = jnp.zeros_like(ce_o)

        em_o[...] += lax.dot_general(oh, en, (((0,), (0,)), ((), ())),
                                     preferred_element_type=F32, precision=_PHX)
        ce_o[...] += lax.dot_general(oh, jnp.ones((be, 128), F32),
                                     (((0,), (0,)), ((), ())),
                                     preferred_element_type=F32, precision=_PHX)

    row = lambda i: (i, 0)
    full = lambda i: (0, 0)
    return pl.pallas_call(
        body,
        grid=grid,
        in_specs=[
            pl.BlockSpec((be, 128), row), pl.BlockSpec((be, 128), row),
            pl.BlockSpec((be, 128), row), pl.BlockSpec((be, 1), row),
            pl.BlockSpec((128, 512), full), pl.BlockSpec((128, 512), full),
            pl.BlockSpec((128, 512), full), pl.BlockSpec((16, 512), full),
            pl.BlockSpec((512, 128), full), pl.BlockSpec((1, 128), full),
        ],
        out_specs=[pl.BlockSpec((be, 128), row),
                   pl.BlockSpec((16, 128), full),
                   pl.BlockSpec((16, 128), full)],
        out_shape=[jax.ShapeDtypeStruct((epad, 128), F32),
                   jax.ShapeDtypeStruct((16, 128), F32),
                   jax.ShapeDtypeStruct((16, 128), F32)],
        compiler_params=pltpu.CompilerParams(
            dimension_semantics=("arbitrary",)),
        interpret=_INTERP,
    )(xg, xcg, e, br, w1r, w1c, w1e, uue, w2, b2)


def _node_mlp_call(x, asum, cnt, bt, wn1x, wn1a, uun, wn2, bn2, bn):
    n = x.shape[0]
    grid = (n // bn,)

    def body(x_r, a_r, cnt_r, bt_r, w1x_r, w1a_r, uun_r, w2_r, b2_r,
             xn_o, xm_o, cb_o):
        inv = 1.0 / jnp.maximum(cnt_r[:, 8:9], 1.0)
        agg = a_r[...] * inv
        oh = (bt_r[...] == lax.broadcasted_iota(jnp.int32, (bn, 16), 1)
              ).astype(F32)
        pre = (jnp.dot(x_r[...], w1x_r[...], preferred_element_type=F32, precision=_PHI)
               + jnp.dot(agg, w1a_r[...], preferred_element_type=F32, precision=_PHI)
               + jnp.dot(oh, uun_r[...], preferred_element_type=F32,
                         precision=_PHX))
        act = jnp.maximum(pre, 0.0)
        xn = jnp.dot(act, w2_r[...], preferred_element_type=F32, precision=_PHI) + b2_r[...]
        xn_o[...] = xn

        @pl.when(pl.program_id(0) == 0)
        def _():
            xm_o[...] = jnp.zeros_like(xm_o)
            cb_o[...] = jnp.zeros_like(cb_o)

        xm_o[...] += lax.dot_general(oh, xn, (((0,), (0,)), ((), ())),
                                     preferred_element_type=F32, precision=_PHX)
        cb_o[...] += lax.dot_general(oh, jnp.ones((bn, 128), F32),
                                     (((0,), (0,)), ((), ())),
                                     preferred_element_type=F32, precision=_PHX)

    row = lambda i: (i, 0)
    full = lambda i: (0, 0)
    return pl.pallas_call(
        body,
        grid=grid,
        in_specs=[
            pl.BlockSpec((bn, 128), row), pl.BlockSpec((bn, 128), row),
            pl.BlockSpec((bn, 128), row), pl.BlockSpec((bn, 1), row),
            pl.BlockSpec((128, 512), full), pl.BlockSpec((128, 512), full),
            pl.BlockSpec((16, 512), full), pl.BlockSpec((512, 128), full),
            pl.BlockSpec((1, 128), full),
        ],
        out_specs=[pl.BlockSpec((bn, 128), row),
                   pl.BlockSpec((16, 128), full),
                   pl.BlockSpec((16, 128), full)],
        out_shape=[jax.ShapeDtypeStruct((n, 128), F32),
                   jax.ShapeDtypeStruct((16, 128), F32),
                   jax.ShapeDtypeStruct((16, 128), F32)],
        compiler_params=pltpu.CompilerParams(
            dimension_semantics=("arbitrary",)),
        interpret=_INTERP,
    )(x, asum, cnt, bt, wn1x, wn1a, uun, wn2, bn2)


def _glob_mlp_call(uh, xm_sum, cb, em_sum, ce, wg1, bg1,
                   wg2, bg2, w1ue, b1e, w1un, b1n):
    def body(uh_r, xms_r, cb_r, ems_r, ce_r, w1_r, b1_r,
             w2_r, b2_r, wue_r, be_r, wun_r, bn_r, uh_o, uue_o, uun_o):
        xm = xms_r[...] / jnp.maximum(cb_r[...], 1.0)
        em = ems_r[...] / jnp.maximum(ce_r[...], 1.0)
        h = jnp.concatenate([uh_r[...], xm, em], axis=-1)
        pre = jnp.dot(h, w1_r[...], preferred_element_type=F32,
                      precision=_PHI) + b1_r[...]
        act = jnp.maximum(pre, 0.0)
        uhn = jnp.dot(act, w2_r[...], preferred_element_type=F32, precision=_PHI) + b2_r[...]
        uh_o[...] = uhn
        uue_o[...] = jnp.dot(uhn, wue_r[...], preferred_element_type=F32, precision=_PHI) + be_r[...]
        uun_o[...] = jnp.dot(uhn, wun_r[...], preferred_element_type=F32, precision=_PHI) + bn_r[...]

    z = lambda i: (0, 0)
    return pl.pallas_call(
        body,
        grid=(1,),
        in_specs=[
            pl.BlockSpec((16, 128), z), pl.BlockSpec((16, 128), z),
            pl.BlockSpec((16, 128), z), pl.BlockSpec((16, 128), z),
            pl.BlockSpec((16, 128), z),
            pl.BlockSpec((384, 512), z), pl.BlockSpec((1, 512), z),
            pl.BlockSpec((512, 128), z), pl.BlockSpec((1, 128), z),
            pl.BlockSpec((128, 512), z), pl.BlockSpec((1, 512), z),
            pl.BlockSpec((128, 512), z), pl.BlockSpec((1, 512), z),
        ],
        out_specs=[pl.BlockSpec((16, 128), z), pl.BlockSpec((16, 512), z),
                   pl.BlockSpec((16, 512), z)],
        out_shape=[jax.ShapeDtypeStruct((16, 128), F32),
                   jax.ShapeDtypeStruct((16, 512), F32),
                   jax.ShapeDtypeStruct((16, 512), F32)],
        interpret=_INTERP,
    )(uh, xm_sum, cb, em_sum, ce, wg1, bg1, wg2, bg2,
      w1ue, b1e, w1un, b1n)


def _head_call(xm_sum, cb, uh, wa, ba, wc1, bc1, wc2t, bc2):
    def body(xms_r, cb_r, uh_r, wa_r, ba_r, wc1_r, bc1_r, wc2_r, bc2_r,
             lg_o, val_o):
        xm = xms_r[...] / jnp.maximum(cb_r[...], 1.0)
        lg_o[...] = jnp.dot(xm, wa_r[...], preferred_element_type=F32, precision=_PHI) + ba_r[...]
        h = jnp.maximum(
            jnp.dot(uh_r[...], wc1_r[...], preferred_element_type=F32, precision=_PHI)
            + bc1_r[...], 0.0)
        v8 = jnp.dot(h, wc2_r[...], preferred_element_type=F32, precision=_PHI)
        val_o[...] = v8[:, 0:1] + bc2_r[...]

    z = lambda i: (0, 0)
    return pl.pallas_call(
        body,
        grid=(1,),
        in_specs=[
            pl.BlockSpec((16, 128), z), pl.BlockSpec((16, 128), z),
            pl.BlockSpec((16, 128), z),
            pl.BlockSpec((128, 8), z), pl.BlockSpec((1, 8), z),
            pl.BlockSpec((128, 128), z), pl.BlockSpec((1, 128), z),
            pl.BlockSpec((128, 8), z), pl.BlockSpec((1, 1), z),
        ],
        out_specs=[pl.BlockSpec((16, 8), z), pl.BlockSpec((16, 1), z)],
        out_shape=[jax.ShapeDtypeStruct((16, 8), F32),
                   jax.ShapeDtypeStruct((16, 1), F32)],
        interpret=_INTERP,
    )(xm_sum, cb, uh, wa, ba, wc1, bc1, wc2t, bc2)


def _r2(v):
    return v.reshape(1, -1)


def kernel(input, node_embedding, edge_attr, edge_index, u, batch, params):
    p = params
    n, d = input.shape
    e_n = edge_attr.shape[0]
    g = u.shape[0]
    h = 8
    bn = 1000
    be = 2048
    epad = 163840

    row = edge_index[0]
    col = edge_index[1]
    row_p = jnp.concatenate([row, jnp.zeros((epad - e_n,), jnp.int32)])
    col_p = jnp.concatenate([col, jnp.zeros((epad - e_n,), jnp.int32)])
    row2d = row_p.reshape(-1, 128)
    col2d = col_p.reshape(-1, 128)
    half = n // 2
    dump = half + (jnp.arange(epad, dtype=jnp.int32) & 7)
    in0 = col_p < half
    col_sc = jnp.stack([jnp.where(in0, col_p, dump),
                        jnp.where(in0, dump, col_p - half)]).reshape(2, -1, 128)
    ea_p = jnp.concatenate([edge_attr, jnp.zeros((epad - e_n, 16), F32)])
    batch2 = batch.reshape(-1, 1)
    zeros_n128 = jnp.zeros((n, 128), F32)

    lane = jnp.arange(128)
    sel = (lane[:, None] // 16 == jnp.arange(h)[None, :]).astype(F32)
    rep = sel.T

    wf = p["fusion"]["w"]
    wqkvs = jnp.concatenate(
        [p["q"]["w"], p["k"]["w"], p["v"]["w"], p["skip"]["w"]], axis=1)
    bqkvs = jnp.concatenate(
        [p["q"]["b"], p["k"]["b"], p["v"]["b"], p["skip"]["b"]]).reshape(1, -1)

    x0, q, k, v, skip = _node_prep_call(
        input, node_embedding,
        p["in_lin"]["w"], _r2(p["in_lin"]["b"]),
        p["emb_lin"]["w"], _r2(p["emb_lin"]["b"]),
        wf, _r2(p["fusion"]["b"]),
        _r2(p["ln1_g"]), _r2(p["ln1_b"]), wqkvs, bqkvs, bn)

    kg = _sc_gather_rows(k, row2d, 128, epad)
    vg = _sc_gather_rows(v, row2d, 128, epad)
    qg = _sc_gather_rows(q, col2d, 128, epad)
    br = _sc_gather_elems(batch, row2d, epad)

    contrib, dencnt_e, e0 = _attn_edge_call(
        kg, vg, qg, ea_p, p["e"]["w"], _r2(p["e"]["b"]),
        p["edge_enc"]["w"], _r2(p["edge_enc"]["b"]), sel, rep, e_n, be)

    msg = _sc_scatter_add(contrib, col_sc, n, zeros_n128)
    dencnt = _sc_scatter_add(dencnt_e, col_sc, n, zeros_n128)

    x = _attn_node_call(
        x0, skip, msg, dencnt, rep,
        _r2(p["ln2_g"]), _r2(p["ln2_b"]),
        p["ff1"]["w"], _r2(p["ff1"]["b"]), p["ff2"]["w"], _r2(p["ff2"]["b"]),
        _r2(p["fn_g"]), _r2(p["fn_b"]), bn)

    m0 = p["meta"][0]
    uh, uue, uun = _glob_prep_call(
        u, p["glob_enc"]["w"], _r2(p["glob_enc"]["b"]),
        m0["e1"]["w"][384:512], _r2(m0["e1"]["b"]),
        m0["n1"]["w"][256:384], _r2(m0["n1"]["b"]))

    e = e0
    br2 = br.reshape(-1, 1)
    xm_sum, cb = None, None
    for li in range(2):
        lp = p["meta"][li]
        xg = _sc_gather_rows(x, row2d, 128, epad)
        xcg = _sc_gather_rows(x, col2d, 128, epad)
        enew, em_sum, ce = _edge_mlp_call(
            xg, xcg, e, br2,
            lp["e1"]["w"][0:128], lp["e1"]["w"][128:256],
            lp["e1"]["w"][256:384], uue,
            lp["e2"]["w"], _r2(lp["e2"]["b"]), e_n, be)
        e = enew
        aggp = _sc_scatter_add(enew, col_sc, n, zeros_n128)
        x, xm_sum, cb = _node_mlp_call(
            x, aggp, dencnt, batch2,
            lp["n1"]["w"][0:128], lp["n1"]["w"][128:256], uun,
            lp["n2"]["w"], _r2(lp["n2"]["b"]), bn)
        if li + 1 < 2:
            nxt = p["meta"][li + 1]
            wue, be1 = nxt["e1"]["w"][384:512], _r2(nxt["e1"]["b"])
            wun, bn1 = nxt["n1"]["w"][256:384], _r2(nxt["n1"]["b"])
        else:
            wue, be1 = jnp.zeros((128, 512), F32), jnp.zeros((1, 512), F32)
            wun, bn1 = jnp.zeros((128, 512), F32), jnp.zeros((1, 512), F32)
        uh, uue, uun = _glob_mlp_call(
            uh, xm_sum, cb, em_sum, ce,
            lp["g1"]["w"], _r2(lp["g1"]["b"]),
            lp["g2"]["w"], _r2(lp["g2"]["b"]), wue, be1, wun, bn1)

    logits, value = _head_call(
        xm_sum, cb, uh, p["actor"]["w"], _r2(p["actor"]["b"]),
        p["c1"]["w"], _r2(p["c1"]["b"]),
        jnp.pad(p["c2"]["w"], ((0, 0), (0, 7))), _r2(p["c2"]["b"]))
    return logits, value


# speedup vs baseline: 6.0121x; 1.0371x over previous
import functools

import jax
import jax.numpy as jnp
from jax import lax
from jax.experimental import pallas as pl
from jax.experimental.pallas import tpu as pltpu
from jax.experimental.pallas import tpu_sc as plsc

F32 = jnp.float32
_PHI = jax.lax.Precision.DEFAULT
_PHX = jax.lax.Precision.HIGHEST
_INTERP = False


_NC, _NS, _L = 2, 16, 16
_NW = _NC * _NS
_GRP = 8
_SUB = 4


def _sc_mesh():
    return plsc.VectorSubcoreMesh(core_axis_name="c", subcore_axis_name="s")


def _sc_gather_rows(table, idx2d, width, out_rows):
    R = idx2d.shape[0]
    per_w = R // _NW
    n_steps = per_w // _GRP
    cs = _SUB * 128

    @functools.partial(
        pl.kernel,
        out_type=jax.ShapeDtypeStruct((out_rows, width), F32),
        mesh=_sc_mesh(),
        scratch_types=[
            pltpu.VMEM((_GRP, 128), jnp.int32),
            pltpu.VMEM((cs, width), F32),
            pltpu.SemaphoreType.DMA,
        ],
    )
    def k(table_hbm, idx_hbm, out_hbm, idx_v, dat_v, sem):
        w = lax.axis_index("c") * _NS + lax.axis_index("s")
        row0 = w * per_w

        def body(t, _):
            r0 = row0 + t * _GRP
            pltpu.sync_copy(idx_hbm.at[pl.ds(r0, _GRP)], idx_v)
            for half in range(_GRP // _SUB):
                cps = [pltpu.async_copy(
                    table_hbm.at[idx_v.at[half * _SUB + j]],
                    dat_v.at[pl.ds(j * 128, 128)], sem)
                    for j in range(_SUB)]
                for cp in cps:
                    cp.wait()
                pltpu.sync_copy(
                    dat_v, out_hbm.at[pl.ds((r0 + half * _SUB) * 128, cs)])
            return 0

        lax.fori_loop(0, n_steps, body, 0)

    return k(table, idx2d)


def _sc_gather_elems(table, idx2d, out_rows):
    R = idx2d.shape[0]
    per_w = R // _NW
    n_steps = per_w // _GRP
    cs = _GRP * 128

    @functools.partial(
        pl.kernel,
        out_type=jax.ShapeDtypeStruct((out_rows,), jnp.int32),
        mesh=_sc_mesh(),
        scratch_types=[
            pltpu.VMEM((_GRP, 128), jnp.int32),
            pltpu.VMEM((cs,), jnp.int32),
            pltpu.SemaphoreType.DMA,
        ],
    )
    def k(table_hbm, idx_hbm, out_hbm, idx_v, dat_v, sem):
        w = lax.axis_index("c") * _NS + lax.axis_index("s")
        row0 = w * per_w

        def body(t, _):
            r0 = row0 + t * _GRP
            pltpu.sync_copy(idx_hbm.at[pl.ds(r0, _GRP)], idx_v)
            cps = [pltpu.async_copy(
                table_hbm.at[idx_v.at[j]],
                dat_v.at[pl.ds(j * 128, 128)], sem) for j in range(_GRP)]
            for cp in cps:
                cp.wait()
            pltpu.sync_copy(dat_v, out_hbm.at[pl.ds(r0 * 128, cs)])
            return 0

        lax.fori_loop(0, n_steps, body, 0)

    return k(table, idx2d)


def _sc_scatter_add(data, idx3d, n_rows, zeros_tab):
    epad, width = data.shape
    R = idx3d.shape[1]
    half_n = n_rows // 2
    tab_rows = half_n + 8
    per_s = R // _NS
    n_steps = per_s // _GRP
    cs = _SUB * 128
    zmain, ztail = (tab_rows // _NS) // 8 * 8, None
    ztail = tab_rows - zmain * _NS
    emain = (half_n // _NS) // 8 * 8
    etail = half_n - emain * _NS

    @functools.partial(
        pl.kernel,
        out_type=jax.ShapeDtypeStruct((n_rows, width), F32),
        mesh=_sc_mesh(),
        scratch_types=[
            pltpu.VMEM((_GRP, 128), jnp.int32),
            pltpu.VMEM((cs, width), F32),
            pltpu.VMEM_SHARED((tab_rows, width), F32),
            pltpu.SemaphoreType.DMA,
        ],
    )
    def k(data_hbm, idx_hbm, zeros_hbm, out_hbm, idx_v, dat_v, shared, sem):
        c = lax.axis_index("c")
        s = lax.axis_index("s")
        pltpu.sync_copy(zeros_hbm.at[pl.ds(s * zmain, zmain)],
                        shared.at[pl.ds(s * zmain, zmain)])
        if ztail:
            @pl.when(s == _NS - 1)
            def _():
                pltpu.sync_copy(zeros_hbm.at[pl.ds(_NS * zmain, ztail)],
                                shared.at[pl.ds(_NS * zmain, ztail)])
        plsc.subcore_barrier()
        row0 = s * per_s

        def body(t, _):
            r0 = row0 + t * _GRP
            pltpu.sync_copy(idx_hbm.at[c, pl.ds(r0, _GRP)], idx_v)
            for h in range(_GRP // _SUB):
                pltpu.sync_copy(
                    data_hbm.at[pl.ds((r0 + h * _SUB) * 128, cs)], dat_v)
                cps = [pltpu.async_copy(dat_v.at[pl.ds(j * 128, 128)],
                                        shared.at[idx_v.at[h * _SUB + j]],
                                        sem, add=True)
                       for j in range(_SUB)]
                for cp in cps:
                    cp.wait()
            return 0

        lax.fori_loop(0, n_steps, body, 0)
        plsc.subcore_barrier()
        pltpu.sync_copy(shared.at[pl.ds(s * emain, emain)],
                        out_hbm.at[pl.ds(c * half_n + s * emain, emain)])
        if etail:
            @pl.when(s == _NS - 1)
            def _():
                pltpu.sync_copy(
                    shared.at[pl.ds(_NS * emain, etail)],
                    out_hbm.at[pl.ds(c * half_n + _NS * emain, etail)])

    return k(data, idx3d, zeros_tab)


def _ln(x, g, b):
    m = jnp.mean(x, axis=-1, keepdims=True)
    d = x - m
    v = jnp.mean(d * d, axis=-1, keepdims=True)
    return d / jnp.sqrt(v + 1e-5) * g + b


def _node_prep_call(inp, emb, wi, bi, we, be, wf, bf, g1, b1,
                    wqkvs, bqkvs, bn):
    n = inp.shape[0]
    grid = (n // bn,)

    def body(inp_r, emb_r, wi_r, bi_r, we_r, be_r, wf_r, bf_r,
             g1_r, b1_r, wq_r, bq_r, x0_o, q_o, k_o, v_o, s_o):
        a = jnp.dot(inp_r[...], wi_r[...], preferred_element_type=F32, precision=_PHI) + bi_r[...]
        ne = jnp.dot(emb_r[...], we_r[...], preferred_element_type=F32, precision=_PHI) + be_r[...]
        comb = jnp.concatenate([ne, a, ne], axis=-1)
        x0 = jnp.dot(comb, wf_r[...], preferred_element_type=F32,
                     precision=_PHI) + bf_r[...]
        xn = _ln(x0, g1_r[...], b1_r[...])
        qkvs = jnp.dot(xn, wq_r[...], preferred_element_type=F32, precision=_PHI) + bq_r[...]
        x0_o[...] = x0
        q_o[...] = qkvs[:, 0:128]
        k_o[...] = qkvs[:, 128:256]
        v_o[...] = qkvs[:, 256:384]
        s_o[...] = qkvs[:, 384:512]

    row = lambda i: (i, 0)
    full = lambda i: (0, 0)
    oshape = jax.ShapeDtypeStruct((n, 128), F32)
    return pl.pallas_call(
        body,
        grid=grid,
        in_specs=[
            pl.BlockSpec((bn, 128), row), pl.BlockSpec((bn, 128), row),
            pl.BlockSpec((128, 128), full), pl.BlockSpec((1, 128), full),
            pl.BlockSpec((128, 128), full), pl.BlockSpec((1, 128), full),
            pl.BlockSpec((384, 128), full),
            pl.BlockSpec((1, 128), full), pl.BlockSpec((1, 128), full),
            pl.BlockSpec((1, 128), full),
            pl.BlockSpec((128, 512), full), pl.BlockSpec((1, 512), full),
        ],
        out_specs=[pl.BlockSpec((bn, 128), row)] * 5,
        out_shape=[oshape] * 5,
        interpret=_INTERP,
    )(inp, emb, wi, bi, we, be, wf, bf, g1, b1, wqkvs, bqkvs)


def _attn_edge_call(kg, vg, qg, ea, watt, batt, wenc, benc, sel, rep,
                    n_edges, be):
    epad = kg.shape[0]
    grid = (epad // be,)

    def body(kg_r, vg_r, qg_r, ea_r, watt_r, batt_r, wenc_r, benc_r,
             sel_r, rep_r, contrib_o, dencnt_o, e0_o):
        i = pl.program_id(0)
        rows = lax.broadcasted_iota(jnp.int32, (be, 1), 0) + i * be
        mask = (rows < n_edges).astype(F32)
        ep = jnp.dot(ea_r[...], watt_r[...], preferred_element_type=F32, precision=_PHI) + batt_r[...]
        ke = kg_r[...] + ep
        ve = vg_r[...] + ep
        alpha = jnp.dot(qg_r[...] * ke, sel_r[...],
                        preferred_element_type=F32, precision=_PHX) * 0.25
        ex = jnp.exp(alpha) * mask
        contrib_o[...] = ve * jnp.dot(ex, rep_r[...],
                                      preferred_element_type=F32,
                                      precision=_PHX)
        dencnt_o[...] = jnp.concatenate(
            [ex, jnp.broadcast_to(mask, (be, 8)), jnp.zeros((be, 112), F32)],
            axis=-1)
        e0_o[...] = (jnp.dot(ea_r[...], wenc_r[...],
                             preferred_element_type=F32, precision=_PHI) + benc_r[...]) * mask

    row = lambda i: (i, 0)
    full = lambda i: (0, 0)
    return pl.pallas_call(
        body,
        grid=grid,
        in_specs=[
            pl.BlockSpec((be, 128), row), pl.BlockSpec((be, 128), row),
            pl.BlockSpec((be, 128), row), pl.BlockSpec((be, 16), row),
            pl.BlockSpec((16, 128), full), pl.BlockSpec((1, 128), full),
            pl.BlockSpec((16, 128), full), pl.BlockSpec((1, 128), full),
            pl.BlockSpec((128, 8), full), pl.BlockSpec((8, 128), full),
        ],
        out_specs=[pl.BlockSpec((be, 128), row), pl.BlockSpec((be, 128), row),
                   pl.BlockSpec((be, 128), row)],
        out_shape=[jax.ShapeDtypeStruct((epad, 128), F32),
                   jax.ShapeDtypeStruct((epad, 128), F32),
                   jax.ShapeDtypeStruct((epad, 128), F32)],
        interpret=_INTERP,
    )(kg, vg, qg, ea, watt, batt, wenc, benc, sel, rep)


def _attn_node_call(x0, skip, msg, dencnt, rep, g2, b2, wf1, bf1,
                    wf2, bf2, fng, fnb, bn):
    n = x0.shape[0]
    grid = (n // bn,)

    def body(x0_r, sk_r, m_r, d_r, rep_r, g2_r, b2_r,
             wf1_r, bf1_r, wf2_r, bf2_r, fng_r, fnb_r, x_o):
        den = jnp.dot(d_r[:, 0:8], rep_r[...],
                      preferred_element_type=F32, precision=_PHX) + 1e-16
        x1 = x0_r[...] + m_r[...] / den + sk_r[...]
        xn = _ln(x1, g2_r[...], b2_r[...])
        h = jnp.maximum(
            jnp.dot(xn, wf1_r[...], preferred_element_type=F32, precision=_PHI) + bf1_r[...], 0.0)
        ff = jnp.dot(h, wf2_r[...], preferred_element_type=F32, precision=_PHI) + bf2_r[...]
        x_o[...] = _ln(x1 + ff, fng_r[...], fnb_r[...])

    row = lambda i: (i, 0)
    full = lambda i: (0, 0)
    return pl.pallas_call(
        body,
        grid=grid,
        in_specs=[
            pl.BlockSpec((bn, 128), row), pl.BlockSpec((bn, 128), row),
            pl.BlockSpec((bn, 128), row), pl.BlockSpec((bn, 128), row),
            pl.BlockSpec((8, 128), full),
            pl.BlockSpec((1, 128), full), pl.BlockSpec((1, 128), full),
            pl.BlockSpec((128, 512), full), pl.BlockSpec((1, 512), full),
            pl.BlockSpec((512, 128), full), pl.BlockSpec((1, 128), full),
            pl.BlockSpec((1, 128), full), pl.BlockSpec((1, 128), full),
        ],
        out_specs=[pl.BlockSpec((bn, 128), row)],
        out_shape=[jax.ShapeDtypeStruct((n, 128), F32)],
        interpret=_INTERP,
    )(x0, skip, msg, dencnt, rep, g2, b2, wf1, bf1, wf2, bf2, fng, fnb)[0]


def _glob_prep_call(u, wg, bg, w1ue, b1e, w1un, b1n):
    def body(u_r, wg_r, bg_r, wue_r, be_r, wun_r, bn_r, uh_o, uue_o, uun_o):
        uh = jnp.dot(u_r[...], wg_r[...], preferred_element_type=F32,
                     precision=_PHI) + bg_r[...]
        uh_o[...] = uh
        uue_o[...] = jnp.dot(uh, wue_r[...], preferred_element_type=F32, precision=_PHI) + be_r[...]
        uun_o[...] = jnp.dot(uh, wun_r[...], preferred_element_type=F32, precision=_PHI) + bn_r[...]

    full = lambda: (0, 0)
    return pl.pallas_call(
        body,
        grid=(1,),
        in_specs=[
            pl.BlockSpec((16, 16), lambda i: (0, 0)),
            pl.BlockSpec((16, 128), lambda i: (0, 0)),
            pl.BlockSpec((1, 128), lambda i: (0, 0)),
            pl.BlockSpec((128, 512), lambda i: (0, 0)),
            pl.BlockSpec((1, 512), lambda i: (0, 0)),
            pl.BlockSpec((128, 512), lambda i: (0, 0)),
            pl.BlockSpec((1, 512), lambda i: (0, 0)),
        ],
        out_specs=[pl.BlockSpec((16, 128), lambda i: (0, 0)),
                   pl.BlockSpec((16, 512), lambda i: (0, 0)),
                   pl.BlockSpec((16, 512), lambda i: (0, 0))],
        out_shape=[jax.ShapeDtypeStruct((16, 128), F32),
                   jax.ShapeDtypeStruct((16, 512), F32),
                   jax.ShapeDtypeStruct((16, 512), F32)],
        interpret=_INTERP,
    )(u, wg, bg, w1ue, b1e, w1un, b1n)


def _edge_mlp_call(xg, xcg, e, br, w1r, w1c, w1e, uue, w2, b2, n_edges, be):
    epad = xg.shape[0]
    grid = (epad // be,)

    def body(xg_r, xcg_r, e_r, br_r, w1r_r, w1c_r, w1e_r, uue_r, w2_r, b2_r,
             enew_o, em_o, ce_o):
        i = pl.program_id(0)
        rows = lax.broadcasted_iota(jnp.int32, (be, 1), 0) + i * be
        mask = (rows < n_edges).astype(F32)
        oh = (br_r[...] == lax.broadcasted_iota(jnp.int32, (be, 16), 1)
              ).astype(F32) * mask
        pre = (jnp.dot(xg_r[...], w1r_r[...], preferred_element_type=F32, precision=_PHI)
               + jnp.dot(xcg_r[...], w1c_r[...], preferred_element_type=F32, precision=_PHI)
               + jnp.dot(e_r[...], w1e_r[...], preferred_element_type=F32, precision=_PHI)
               + jnp.dot(oh, uue_r[...], preferred_element_type=F32,
                         precision=_PHX))
        act = jnp.maximum(pre, 0.0)
        en = (jnp.dot(act, w2_r[...], preferred_element_type=F32, precision=_PHI)
              + b2_r[...]) * mask
        enew_o[...] = en

        @pl.when(i == 0)
        def _():
            em_o[...] = jnp.zeros_like(em_o)
            ce_o[...] = jnp.zeros_like(ce_o)

        em_o[...] += lax.dot_general(oh, en, (((0,), (0,)), ((), ())),
                                     preferred_element_type=F32, precision=_PHX)
        ce_o[...] += lax.dot_general(oh, jnp.ones((be, 128), F32),
                                     (((0,), (0,)), ((), ())),
                                     preferred_element_type=F32, precision=_PHX)

    row = lambda i: (i, 0)
    full = lambda i: (0, 0)
    return pl.pallas_call(
        body,
        grid=grid,
        in_specs=[
            pl.BlockSpec((be, 128), row), pl.BlockSpec((be, 128), row),
            pl.BlockSpec((be, 128), row), pl.BlockSpec((be, 1), row),
            pl.BlockSpec((128, 512), full), pl.BlockSpec((128, 512), full),
            pl.BlockSpec((128, 512), full), pl.BlockSpec((16, 512), full),
            pl.BlockSpec((512, 128), full), pl.BlockSpec((1, 128), full),
        ],
        out_specs=[pl.BlockSpec((be, 128), row),
                   pl.BlockSpec((16, 128), full),
                   pl.BlockSpec((16, 128), full)],
        out_shape=[jax.ShapeDtypeStruct((epad, 128), F32),
                   jax.ShapeDtypeStruct((16, 128), F32),
                   jax.ShapeDtypeStruct((16, 128), F32)],
        compiler_params=pltpu.CompilerParams(
            dimension_semantics=("arbitrary",)),
        interpret=_INTERP,
    )(xg, xcg, e, br, w1r, w1c, w1e, uue, w2, b2)


def _node_mlp_call(x, asum, cnt, bt, wn1x, wn1a, uun, wn2, bn2, bn):
    n = x.shape[0]
    grid = (n // bn,)

    def body(x_r, a_r, cnt_r, bt_r, w1x_r, w1a_r, uun_r, w2_r, b2_r,
             xn_o, xm_o, cb_o):
        inv = 1.0 / jnp.maximum(cnt_r[:, 8:9], 1.0)
        agg = a_r[...] * inv
        oh = (bt_r[...] == lax.broadcasted_iota(jnp.int32, (bn, 16), 1)
              ).astype(F32)
        pre = (jnp.dot(x_r[...], w1x_r[...], preferred_element_type=F32, precision=_PHI)
               + jnp.dot(agg, w1a_r[...], preferred_element_type=F32, precision=_PHI)
               + jnp.dot(oh, uun_r[...], preferred_element_type=F32,
                         precision=_PHX))
        act = jnp.maximum(pre, 0.0)
        xn = jnp.dot(act, w2_r[...], preferred_element_type=F32, precision=_PHI) + b2_r[...]
        xn_o[...] = xn

        @pl.when(pl.program_id(0) == 0)
        def _():
            xm_o[...] = jnp.zeros_like(xm_o)
            cb_o[...] = jnp.zeros_like(cb_o)

        xm_o[...] += lax.dot_general(oh, xn, (((0,), (0,)), ((), ())),
                                     preferred_element_type=F32, precision=_PHX)
        cb_o[...] += lax.dot_general(oh, jnp.ones((bn, 128), F32),
                                     (((0,), (0,)), ((), ())),
                                     preferred_element_type=F32, precision=_PHX)

    row = lambda i: (i, 0)
    full = lambda i: (0, 0)
    return pl.pallas_call(
        body,
        grid=grid,
        in_specs=[
            pl.BlockSpec((bn, 128), row), pl.BlockSpec((bn, 128), row),
            pl.BlockSpec((bn, 128), row), pl.BlockSpec((bn, 1), row),
            pl.BlockSpec((128, 512), full), pl.BlockSpec((128, 512), full),
            pl.BlockSpec((16, 512), full), pl.BlockSpec((512, 128), full),
            pl.BlockSpec((1, 128), full),
        ],
        out_specs=[pl.BlockSpec((bn, 128), row),
                   pl.BlockSpec((16, 128), full),
                   pl.BlockSpec((16, 128), full)],
        out_shape=[jax.ShapeDtypeStruct((n, 128), F32),
                   jax.ShapeDtypeStruct((16, 128), F32),
                   jax.ShapeDtypeStruct((16, 128), F32)],
        compiler_params=pltpu.CompilerParams(
            dimension_semantics=("arbitrary",)),
        interpret=_INTERP,
    )(x, asum, cnt, bt, wn1x, wn1a, uun, wn2, bn2)


def _glob_mlp_call(uh, xm_sum, cb, em_sum, ce, wg1, bg1,
                   wg2, bg2, w1ue, b1e, w1un, b1n):
    def body(uh_r, xms_r, cb_r, ems_r, ce_r, w1_r, b1_r,
             w2_r, b2_r, wue_r, be_r, wun_r, bn_r, uh_o, uue_o, uun_o):
        xm = xms_r[...] / jnp.maximum(cb_r[...], 1.0)
        em = ems_r[...] / jnp.maximum(ce_r[...], 1.0)
        h = jnp.concatenate([uh_r[...], xm, em], axis=-1)
        pre = jnp.dot(h, w1_r[...], preferred_element_type=F32,
                      precision=_PHI) + b1_r[...]
        act = jnp.maximum(pre, 0.0)
        uhn = jnp.dot(act, w2_r[...], preferred_element_type=F32, precision=_PHI) + b2_r[...]
        uh_o[...] = uhn
        uue_o[...] = jnp.dot(uhn, wue_r[...], preferred_element_type=F32, precision=_PHI) + be_r[...]
        uun_o[...] = jnp.dot(uhn, wun_r[...], preferred_element_type=F32, precision=_PHI) + bn_r[...]

    z = lambda i: (0, 0)
    return pl.pallas_call(
        body,
        grid=(1,),
        in_specs=[
            pl.BlockSpec((16, 128), z), pl.BlockSpec((16, 128), z),
            pl.BlockSpec((16, 128), z), pl.BlockSpec((16, 128), z),
            pl.BlockSpec((16, 128), z),
            pl.BlockSpec((384, 512), z), pl.BlockSpec((1, 512), z),
            pl.BlockSpec((512, 128), z), pl.BlockSpec((1, 128), z),
            pl.BlockSpec((128, 512), z), pl.BlockSpec((1, 512), z),
            pl.BlockSpec((128, 512), z), pl.BlockSpec((1, 512), z),
        ],
        out_specs=[pl.BlockSpec((16, 128), z), pl.BlockSpec((16, 512), z),
                   pl.BlockSpec((16, 512), z)],
        out_shape=[jax.ShapeDtypeStruct((16, 128), F32),
                   jax.ShapeDtypeStruct((16, 512), F32),
                   jax.ShapeDtypeStruct((16, 512), F32)],
        interpret=_INTERP,
    )(uh, xm_sum, cb, em_sum, ce, wg1, bg1, wg2, bg2,
      w1ue, b1e, w1un, b1n)


def _head_call(xm_sum, cb, uh, wa, ba, wc1, bc1, wc2t, bc2):
    def body(xms_r, cb_r, uh_r, wa_r, ba_r, wc1_r, bc1_r, wc2_r, bc2_r,
             lg_o, val_o):
        xm = xms_r[...] / jnp.maximum(cb_r[...], 1.0)
        lg_o[...] = jnp.dot(xm, wa_r[...], preferred_element_type=F32, precision=_PHI) + ba_r[...]
        h = jnp.maximum(
            jnp.dot(uh_r[...], wc1_r[...], preferred_element_type=F32, precision=_PHI)
            + bc1_r[...], 0.0)
        v8 = jnp.dot(h, wc2_r[...], preferred_element_type=F32, precision=_PHI)
        val_o[...] = v8[:, 0:1] + bc2_r[...]

    z = lambda i: (0, 0)
    return pl.pallas_call(
        body,
        grid=(1,),
        in_specs=[
            pl.BlockSpec((16, 128), z), pl.BlockSpec((16, 128), z),
            pl.BlockSpec((16, 128), z),
            pl.BlockSpec((128, 8), z), pl.BlockSpec((1, 8), z),
            pl.BlockSpec((128, 128), z), pl.BlockSpec((1, 128), z),
            pl.BlockSpec((128, 8), z), pl.BlockSpec((1, 1), z),
        ],
        out_specs=[pl.BlockSpec((16, 8), z), pl.BlockSpec((16, 1), z)],
        out_shape=[jax.ShapeDtypeStruct((16, 8), F32),
                   jax.ShapeDtypeStruct((16, 1), F32)],
        interpret=_INTERP,
    )(xm_sum, cb, uh, wa, ba, wc1, bc1, wc2t, bc2)


def _r2(v):
    return v.reshape(1, -1)


def kernel(input, node_embedding, edge_attr, edge_index, u, batch, params):
    p = params
    n, d = input.shape
    e_n = edge_attr.shape[0]
    g = u.shape[0]
    h = 8
    bn = 1000
    be = 2048
    epad = 163840

    row = edge_index[0]
    col = edge_index[1]
    row_p = jnp.concatenate([row, jnp.zeros((epad - e_n,), jnp.int32)])
    col_p = jnp.concatenate([col, jnp.zeros((epad - e_n,), jnp.int32)])
    row2d = row_p.reshape(-1, 128)
    col2d = col_p.reshape(-1, 128)
    half = n // 2
    dump = half + (jnp.arange(epad, dtype=jnp.int32) & 7)
    in0 = col_p < half
    col_sc = jnp.stack([jnp.where(in0, col_p, dump),
                        jnp.where(in0, dump, col_p - half)]).reshape(2, -1, 128)
    ea_p = jnp.concatenate([edge_attr, jnp.zeros((epad - e_n, 16), F32)])
    batch2 = batch.reshape(-1, 1)
    zeros_n128 = jnp.zeros((n, 128), F32)

    lane = jnp.arange(128)
    sel = (lane[:, None] // 16 == jnp.arange(h)[None, :]).astype(F32)
    rep = sel.T

    wf = p["fusion"]["w"]
    wqkvs = jnp.concatenate(
        [p["q"]["w"], p["k"]["w"], p["v"]["w"], p["skip"]["w"]], axis=1)
    bqkvs = jnp.concatenate(
        [p["q"]["b"], p["k"]["b"], p["v"]["b"], p["skip"]["b"]]).reshape(1, -1)

    x0, q, k, v, skip = _node_prep_call(
        input, node_embedding,
        p["in_lin"]["w"], _r2(p["in_lin"]["b"]),
        p["emb_lin"]["w"], _r2(p["emb_lin"]["b"]),
        wf, _r2(p["fusion"]["b"]),
        _r2(p["ln1_g"]), _r2(p["ln1_b"]), wqkvs, bqkvs, bn)

    kg = _sc_gather_rows(k, row2d, 128, epad)
    vg = _sc_gather_rows(v, row2d, 128, epad)
    qg = _sc_gather_rows(q, col2d, 128, epad)
    br = _sc_gather_elems(batch, row2d, epad)

    contrib, dencnt_e, e0 = _attn_edge_call(
        kg, vg, qg, ea_p, p["e"]["w"], _r2(p["e"]["b"]),
        p["edge_enc"]["w"], _r2(p["edge_enc"]["b"]), sel, rep, e_n, be)

    msg = _sc_scatter_add(contrib, col_sc, n, zeros_n128)
    dencnt = _sc_scatter_add(dencnt_e, col_sc, n, zeros_n128)

    x = _attn_node_call(
        x0, skip, msg, dencnt, rep,
        _r2(p["ln2_g"]), _r2(p["ln2_b"]),
        p["ff1"]["w"], _r2(p["ff1"]["b"]), p["ff2"]["w"], _r2(p["ff2"]["b"]),
        _r2(p["fn_g"]), _r2(p["fn_b"]), bn)

    m0 = p["meta"][0]
    uh, uue, uun = _glob_prep_call(
        u, p["glob_enc"]["w"], _r2(p["glob_enc"]["b"]),
        m0["e1"]["w"][384:512], _r2(m0["e1"]["b"]),
        m0["n1"]["w"][256:384], _r2(m0["n1"]["b"]))

    e = e0
    br2 = br.reshape(-1, 1)
    xm_sum, cb = None, None
    for li in range(2):
        lp = p["meta"][li]
        xg = _sc_gather_rows(x, row2d, 128, epad)
        xcg = _sc_gather_rows(x, col2d, 128, epad)
        enew, em_sum, ce = _edge_mlp_call(
            xg, xcg, e, br2,
            lp["e1"]["w"][0:128], lp["e1"]["w"][128:256],
            lp["e1"]["w"][256:384], uue,
            lp["e2"]["w"], _r2(lp["e2"]["b"]), e_n, be)
        e = enew
        aggp = _sc_scatter_add(enew, col_sc, n, zeros_n128)
        x, xm_sum, cb = _node_mlp_call(
            x, aggp, dencnt, batch2,
            lp["n1"]["w"][0:128], lp["n1"]["w"][128:256], uun,
            lp["n2"]["w"], _r2(lp["n2"]["b"]), bn)
        if li + 1 < 2:
            nxt = p["meta"][li + 1]
            wue, be1 = nxt["e1"]["w"][384:512], _r2(nxt["e1"]["b"])
            wun, bn1 = nxt["n1"]["w"][256:384], _r2(nxt["n1"]["b"])
        else:
            wue, be1 = jnp.zeros((128, 512), F32), jnp.zeros((1, 512), F32)
            wun, bn1 = jnp.zeros((128, 512), F32), jnp.zeros((1, 512), F32)
        uh, uue, uun = _glob_mlp_call(
            uh, xm_sum, cb, em_sum, ce,
            lp["g1"]["w"], _r2(lp["g1"]["b"]),
            lp["g2"]["w"], _r2(lp["g2"]["b"]), wue, be1, wun, bn1)

    logits, value = _head_call(
        xm_sum, cb, uh, p["actor"]["w"], _r2(p["actor"]["b"]),
        p["c1"]["w"], _r2(p["c1"]["b"]),
        jnp.pad(p["c2"]["w"], ((0, 0), (0, 7))), _r2(p["c2"]["b"]))
    return logits, value


# speedup vs baseline: 6.0563x; 1.0073x over previous
import functools

import jax
import jax.numpy as jnp
from jax import lax
from jax.experimental import pallas as pl
from jax.experimental.pallas import tpu as pltpu
from jax.experimental.pallas import tpu_sc as plsc

F32 = jnp.float32
_PHI = jax.lax.Precision.DEFAULT
_PHX = jax.lax.Precision.HIGHEST
_INTERP = False


_NC, _NS, _L = 2, 16, 16
_NW = _NC * _NS
_GRP = 8
_SUB = 4


def _sc_mesh():
    return plsc.VectorSubcoreMesh(core_axis_name="c", subcore_axis_name="s")


def _sc_gather_rows(table, idx2d, width, out_rows, dtype=F32):
    R = idx2d.shape[0]
    per_w = R // _NW
    n_steps = per_w // _GRP
    cs = _SUB * 128

    @functools.partial(
        pl.kernel,
        out_type=jax.ShapeDtypeStruct((out_rows, width), dtype),
        mesh=_sc_mesh(),
        scratch_types=[
            pltpu.VMEM((_GRP, 128), jnp.int32),
            pltpu.VMEM((cs, width), dtype),
        ] + [pltpu.SemaphoreType.DMA] * 8,
    )
    def k(table_hbm, idx_hbm, out_hbm, idx_v, dat_v, *sems):
        gsem, osem = sems[0:4], sems[4:8]
        w = lax.axis_index("c") * _NS + lax.axis_index("s")
        row0 = w * per_w

        def quarter(q):
            return dat_v.at[pl.ds(q * 128, 128)]

        def body(t, _):
            r0 = row0 + t * _GRP
            pltpu.sync_copy(idx_hbm.at[pl.ds(r0, _GRP)], idx_v)
            g1 = [pltpu.async_copy(table_hbm.at[idx_v.at[j]],
                                   quarter(j), gsem[j]) for j in range(4)]
            o1 = []
            for j in range(4):
                g1[j].wait()
                o1.append(pltpu.async_copy(
                    quarter(j), out_hbm.at[pl.ds((r0 + j) * 128, 128)],
                    osem[j]))
            g2 = []
            for j in range(4):
                o1[j].wait()
                g2.append(pltpu.async_copy(table_hbm.at[idx_v.at[4 + j]],
                                           quarter(j), gsem[j]))
            o2 = []
            for j in range(4):
                g2[j].wait()
                o2.append(pltpu.async_copy(
                    quarter(j), out_hbm.at[pl.ds((r0 + 4 + j) * 128, 128)],
                    osem[j]))
            for j in range(4):
                o2[j].wait()
            return 0

        lax.fori_loop(0, n_steps, body, 0)

    return k(table, idx2d)


def _sc_gather_elems(table, idx2d, out_rows):
    R = idx2d.shape[0]
    per_w = R // _NW
    n_steps = per_w // _GRP
    cs = _GRP * 128

    @functools.partial(
        pl.kernel,
        out_type=jax.ShapeDtypeStruct((out_rows,), jnp.int32),
        mesh=_sc_mesh(),
        scratch_types=[
            pltpu.VMEM((_GRP, 128), jnp.int32),
            pltpu.VMEM((cs,), jnp.int32),
            pltpu.SemaphoreType.DMA,
        ],
    )
    def k(table_hbm, idx_hbm, out_hbm, idx_v, dat_v, sem):
        w = lax.axis_index("c") * _NS + lax.axis_index("s")
        row0 = w * per_w

        def body(t, _):
            r0 = row0 + t * _GRP
            pltpu.sync_copy(idx_hbm.at[pl.ds(r0, _GRP)], idx_v)
            cps = [pltpu.async_copy(
                table_hbm.at[idx_v.at[j]],
                dat_v.at[pl.ds(j * 128, 128)], sem) for j in range(_GRP)]
            for cp in cps:
                cp.wait()
            pltpu.sync_copy(dat_v, out_hbm.at[pl.ds(r0 * 128, cs)])
            return 0

        lax.fori_loop(0, n_steps, body, 0)

    return k(table, idx2d)


def _sc_scatter_add(data, idx3d, n_rows, zeros_tab):
    epad, width = data.shape
    R = idx3d.shape[1]
    half_n = n_rows // 2
    tab_rows = half_n + 8
    per_s = R // _NS
    n_steps = per_s // _GRP
    cs = _SUB * 128
    zmain, ztail = (tab_rows // _NS) // 8 * 8, None
    ztail = tab_rows - zmain * _NS
    emain = (half_n // _NS) // 8 * 8
    etail = half_n - emain * _NS

    @functools.partial(
        pl.kernel,
        out_type=jax.ShapeDtypeStruct((n_rows, width), F32),
        mesh=_sc_mesh(),
        scratch_types=[
            pltpu.VMEM((_GRP, 128), jnp.int32),
            pltpu.VMEM((cs, width), F32),
            pltpu.VMEM_SHARED((tab_rows, width), F32),
            pltpu.SemaphoreType.DMA,
        ],
    )
    def k(data_hbm, idx_hbm, zeros_hbm, out_hbm, idx_v, dat_v, shared, sem):
        c = lax.axis_index("c")
        s = lax.axis_index("s")
        pltpu.sync_copy(zeros_hbm.at[pl.ds(s * zmain, zmain)],
                        shared.at[pl.ds(s * zmain, zmain)])
        if ztail:
            @pl.when(s == _NS - 1)
            def _():
                pltpu.sync_copy(zeros_hbm.at[pl.ds(_NS * zmain, ztail)],
                                shared.at[pl.ds(_NS * zmain, ztail)])
        plsc.subcore_barrier()
        row0 = s * per_s

        def body(t, _):
            r0 = row0 + t * _GRP
            pltpu.sync_copy(idx_hbm.at[c, pl.ds(r0, _GRP)], idx_v)
            for h in range(_GRP // _SUB):
                pltpu.sync_copy(
                    data_hbm.at[pl.ds((r0 + h * _SUB) * 128, cs)], dat_v)
                cps = [pltpu.async_copy(dat_v.at[pl.ds(j * 128, 128)],
                                        shared.at[idx_v.at[h * _SUB + j]],
                                        sem, add=True)
                       for j in range(_SUB)]
                for cp in cps:
                    cp.wait()
            return 0

        lax.fori_loop(0, n_steps, body, 0)
        plsc.subcore_barrier()
        pltpu.sync_copy(shared.at[pl.ds(s * emain, emain)],
                        out_hbm.at[pl.ds(c * half_n + s * emain, emain)])
        if etail:
            @pl.when(s == _NS - 1)
            def _():
                pltpu.sync_copy(
                    shared.at[pl.ds(_NS * emain, etail)],
                    out_hbm.at[pl.ds(c * half_n + _NS * emain, etail)])

    return k(data, idx3d, zeros_tab)


def _ln(x, g, b):
    m = jnp.mean(x, axis=-1, keepdims=True)
    d = x - m
    v = jnp.mean(d * d, axis=-1, keepdims=True)
    return d / jnp.sqrt(v + 1e-5) * g + b


def _node_prep_call(inp, emb, wi, bi, we, be, wf, bf, g1, b1,
                    wqkvs, bqkvs, bn):
    n = inp.shape[0]
    grid = (n // bn,)

    def body(inp_r, emb_r, wi_r, bi_r, we_r, be_r, wf_r, bf_r,
             g1_r, b1_r, wq_r, bq_r, x0_o, q_o, k_o, v_o, s_o):
        a = jnp.dot(inp_r[...], wi_r[...], preferred_element_type=F32, precision=_PHI) + bi_r[...]
        ne = jnp.dot(emb_r[...], we_r[...], preferred_element_type=F32, precision=_PHI) + be_r[...]
        comb = jnp.concatenate([ne, a, ne], axis=-1)
        x0 = jnp.dot(comb, wf_r[...], preferred_element_type=F32,
                     precision=_PHI) + bf_r[...]
        xn = _ln(x0, g1_r[...], b1_r[...])
        qkvs = jnp.dot(xn, wq_r[...], preferred_element_type=F32, precision=_PHI) + bq_r[...]
        x0_o[...] = x0
        q_o[...] = qkvs[:, 0:128]
        k_o[...] = qkvs[:, 128:256]
        v_o[...] = qkvs[:, 256:384]
        s_o[...] = qkvs[:, 384:512]

    row = lambda i: (i, 0)
    full = lambda i: (0, 0)
    oshape = jax.ShapeDtypeStruct((n, 128), F32)
    return pl.pallas_call(
        body,
        grid=grid,
        in_specs=[
            pl.BlockSpec((bn, 128), row), pl.BlockSpec((bn, 128), row),
            pl.BlockSpec((128, 128), full), pl.BlockSpec((1, 128), full),
            pl.BlockSpec((128, 128), full), pl.BlockSpec((1, 128), full),
            pl.BlockSpec((384, 128), full),
            pl.BlockSpec((1, 128), full), pl.BlockSpec((1, 128), full),
            pl.BlockSpec((1, 128), full),
            pl.BlockSpec((128, 512), full), pl.BlockSpec((1, 512), full),
        ],
        out_specs=[pl.BlockSpec((bn, 128), row)] * 5,
        out_shape=[oshape] * 5,
        interpret=_INTERP,
    )(inp, emb, wi, bi, we, be, wf, bf, g1, b1, wqkvs, bqkvs)


def _attn_edge_call(kg, vg, qg, ea, watt, batt, wenc, benc, sel, rep,
                    n_edges, be):
    epad = kg.shape[0]
    grid = (epad // be,)

    def body(kg_r, vg_r, qg_r, ea_r, watt_r, batt_r, wenc_r, benc_r,
             sel_r, rep_r, contrib_o, dencnt_o, e0_o):
        i = pl.program_id(0)
        rows = lax.broadcasted_iota(jnp.int32, (be, 1), 0) + i * be
        mask = (rows < n_edges).astype(F32)
        ep = jnp.dot(ea_r[...], watt_r[...], preferred_element_type=F32, precision=_PHI) + batt_r[...]
        ke = kg_r[...] + ep
        ve = vg_r[...] + ep
        alpha = jnp.dot(qg_r[...] * ke, sel_r[...],
                        preferred_element_type=F32, precision=_PHX) * 0.25
        ex = jnp.exp(alpha) * mask
        contrib_o[...] = ve * jnp.dot(ex, rep_r[...],
                                      preferred_element_type=F32,
                                      precision=_PHX)
        dencnt_o[...] = jnp.concatenate(
            [ex, jnp.broadcast_to(mask, (be, 8)), jnp.zeros((be, 112), F32)],
            axis=-1)
        e0_o[...] = (jnp.dot(ea_r[...], wenc_r[...],
                             preferred_element_type=F32, precision=_PHI) + benc_r[...]) * mask

    row = lambda i: (i, 0)
    full = lambda i: (0, 0)
    return pl.pallas_call(
        body,
        grid=grid,
        in_specs=[
            pl.BlockSpec((be, 128), row), pl.BlockSpec((be, 128), row),
            pl.BlockSpec((be, 128), row), pl.BlockSpec((be, 16), row),
            pl.BlockSpec((16, 128), full), pl.BlockSpec((1, 128), full),
            pl.BlockSpec((16, 128), full), pl.BlockSpec((1, 128), full),
            pl.BlockSpec((128, 8), full), pl.BlockSpec((8, 128), full),
        ],
        out_specs=[pl.BlockSpec((be, 128), row), pl.BlockSpec((be, 128), row),
                   pl.BlockSpec((be, 128), row)],
        out_shape=[jax.ShapeDtypeStruct((epad, 128), F32),
                   jax.ShapeDtypeStruct((epad, 128), F32),
                   jax.ShapeDtypeStruct((epad, 128), F32)],
        interpret=_INTERP,
    )(kg, vg, qg, ea, watt, batt, wenc, benc, sel, rep)


def _attn_node_call(x0, skip, msg, dencnt, rep, g2, b2, wf1, bf1,
                    wf2, bf2, fng, fnb, bn):
    n = x0.shape[0]
    grid = (n // bn,)

    def body(x0_r, sk_r, m_r, d_r, rep_r, g2_r, b2_r,
             wf1_r, bf1_r, wf2_r, bf2_r, fng_r, fnb_r, x_o):
        den = jnp.dot(d_r[:, 0:8], rep_r[...],
                      preferred_element_type=F32, precision=_PHX) + 1e-16
        x1 = x0_r[...] + m_r[...] / den + sk_r[...]
        xn = _ln(x1, g2_r[...], b2_r[...])
        h = jnp.maximum(
            jnp.dot(xn, wf1_r[...], preferred_element_type=F32, precision=_PHI) + bf1_r[...], 0.0)
        ff = jnp.dot(h, wf2_r[...], preferred_element_type=F32, precision=_PHI) + bf2_r[...]
        x_o[...] = _ln(x1 + ff, fng_r[...], fnb_r[...])

    row = lambda i: (i, 0)
    full = lambda i: (0, 0)
    return pl.pallas_call(
        body,
        grid=grid,
        in_specs=[
            pl.BlockSpec((bn, 128), row), pl.BlockSpec((bn, 128), row),
            pl.BlockSpec((bn, 128), row), pl.BlockSpec((bn, 128), row),
            pl.BlockSpec((8, 128), full),
            pl.BlockSpec((1, 128), full), pl.BlockSpec((1, 128), full),
            pl.BlockSpec((128, 512), full), pl.BlockSpec((1, 512), full),
            pl.BlockSpec((512, 128), full), pl.BlockSpec((1, 128), full),
            pl.BlockSpec((1, 128), full), pl.BlockSpec((1, 128), full),
        ],
        out_specs=[pl.BlockSpec((bn, 128), row)],
        out_shape=[jax.ShapeDtypeStruct((n, 128), F32)],
        interpret=_INTERP,
    )(x0, skip, msg, dencnt, rep, g2, b2, wf1, bf1, wf2, bf2, fng, fnb)[0]


def _glob_prep_call(u, wg, bg, w1ue, b1e, w1un, b1n):
    def body(u_r, wg_r, bg_r, wue_r, be_r, wun_r, bn_r, uh_o, uue_o, uun_o):
        uh = jnp.dot(u_r[...], wg_r[...], preferred_element_type=F32,
                     precision=_PHI) + bg_r[...]
        uh_o[...] = uh
        uue_o[...] = jnp.dot(uh, wue_r[...], preferred_element_type=F32, precision=_PHI) + be_r[...]
        uun_o[...] = jnp.dot(uh, wun_r[...], preferred_element_type=F32, precision=_PHI) + bn_r[...]

    full = lambda: (0, 0)
    return pl.pallas_call(
        body,
        grid=(1,),
        in_specs=[
            pl.BlockSpec((16, 16), lambda i: (0, 0)),
            pl.BlockSpec((16, 128), lambda i: (0, 0)),
            pl.BlockSpec((1, 128), lambda i: (0, 0)),
            pl.BlockSpec((128, 512), lambda i: (0, 0)),
            pl.BlockSpec((1, 512), lambda i: (0, 0)),
            pl.BlockSpec((128, 512), lambda i: (0, 0)),
            pl.BlockSpec((1, 512), lambda i: (0, 0)),
        ],
        out_specs=[pl.BlockSpec((16, 128), lambda i: (0, 0)),
                   pl.BlockSpec((16, 512), lambda i: (0, 0)),
                   pl.BlockSpec((16, 512), lambda i: (0, 0))],
        out_shape=[jax.ShapeDtypeStruct((16, 128), F32),
                   jax.ShapeDtypeStruct((16, 512), F32),
                   jax.ShapeDtypeStruct((16, 512), F32)],
        interpret=_INTERP,
    )(u, wg, bg, w1ue, b1e, w1un, b1n)


def _edge_mlp_call(xg, xcg, e, br, w1r, w1c, w1e, uue, w2, b2, n_edges, be):
    epad = xg.shape[0]
    grid = (epad // be,)

    def body(xg_r, xcg_r, e_r, br_r, w1r_r, w1c_r, w1e_r, uue_r, w2_r, b2_r,
             enew_o, em_o, ce_o):
        i = pl.program_id(0)
        rows = lax.broadcasted_iota(jnp.int32, (be, 1), 0) + i * be
        mask = (rows < n_edges).astype(F32)
        oh = (br_r[...] == lax.broadcasted_iota(jnp.int32, (be, 16), 1)
              ).astype(F32) * mask
        pre = (jnp.dot(xg_r[...], w1r_r[...],
                       preferred_element_type=F32, precision=_PHI)
               + jnp.dot(xcg_r[...], w1c_r[...],
                         preferred_element_type=F32, precision=_PHI)
               + jnp.dot(e_r[...], w1e_r[...], preferred_element_type=F32, precision=_PHI)
               + jnp.dot(oh, uue_r[...], preferred_element_type=F32,
                         precision=_PHX))
        act = jnp.maximum(pre, 0.0)
        en = (jnp.dot(act, w2_r[...], preferred_element_type=F32, precision=_PHI)
              + b2_r[...]) * mask
        enew_o[...] = en

        @pl.when(i == 0)
        def _():
            em_o[...] = jnp.zeros_like(em_o)
            ce_o[...] = jnp.zeros_like(ce_o)

        em_o[...] += lax.dot_general(oh, en, (((0,), (0,)), ((), ())),
                                     preferred_element_type=F32, precision=_PHX)
        ce_o[...] += lax.dot_general(oh, jnp.ones((be, 128), F32),
                                     (((0,), (0,)), ((), ())),
                                     preferred_element_type=F32, precision=_PHX)

    row = lambda i: (i, 0)
    full = lambda i: (0, 0)
    return pl.pallas_call(
        body,
        grid=grid,
        in_specs=[
            pl.BlockSpec((be, 128), row), pl.BlockSpec((be, 128), row),
            pl.BlockSpec((be, 128), row), pl.BlockSpec((be, 1), row),
            pl.BlockSpec((128, 512), full), pl.BlockSpec((128, 512), full),
            pl.BlockSpec((128, 512), full), pl.BlockSpec((16, 512), full),
            pl.BlockSpec((512, 128), full), pl.BlockSpec((1, 128), full),
        ],
        out_specs=[pl.BlockSpec((be, 128), row),
                   pl.BlockSpec((16, 128), full),
                   pl.BlockSpec((16, 128), full)],
        out_shape=[jax.ShapeDtypeStruct((epad, 128), F32),
                   jax.ShapeDtypeStruct((16, 128), F32),
                   jax.ShapeDtypeStruct((16, 128), F32)],
        compiler_params=pltpu.CompilerParams(
            dimension_semantics=("arbitrary",)),
        interpret=_INTERP,
    )(xg, xcg, e, br, w1r, w1c, w1e, uue, w2, b2)


def _node_mlp_call(x, asum, cnt, bt, wn1x, wn1a, uun, wn2, bn2, bn):
    n = x.shape[0]
    grid = (n // bn,)

    def body(x_r, a_r, cnt_r, bt_r, w1x_r, w1a_r, uun_r, w2_r, b2_r,
             xn_o, xm_o, cb_o):
        inv = 1.0 / jnp.maximum(cnt_r[:, 8:9], 1.0)
        agg = a_r[...] * inv
        oh = (bt_r[...] == lax.broadcasted_iota(jnp.int32, (bn, 16), 1)
              ).astype(F32)
        pre = (jnp.dot(x_r[...], w1x_r[...], preferred_element_type=F32, precision=_PHI)
               + jnp.dot(agg, w1a_r[...], preferred_element_type=F32, precision=_PHI)
               + jnp.dot(oh, uun_r[...], preferred_element_type=F32,
                         precision=_PHX))
        act = jnp.maximum(pre, 0.0)
        xn = jnp.dot(act, w2_r[...], preferred_element_type=F32, precision=_PHI) + b2_r[...]
        xn_o[...] = xn

        @pl.when(pl.program_id(0) == 0)
        def _():
            xm_o[...] = jnp.zeros_like(xm_o)
            cb_o[...] = jnp.zeros_like(cb_o)

        xm_o[...] += lax.dot_general(oh, xn, (((0,), (0,)), ((), ())),
                                     preferred_element_type=F32, precision=_PHX)
        cb_o[...] += lax.dot_general(oh, jnp.ones((bn, 128), F32),
                                     (((0,), (0,)), ((), ())),
                                     preferred_element_type=F32, precision=_PHX)

    row = lambda i: (i, 0)
    full = lambda i: (0, 0)
    return pl.pallas_call(
        body,
        grid=grid,
        in_specs=[
            pl.BlockSpec((bn, 128), row), pl.BlockSpec((bn, 128), row),
            pl.BlockSpec((bn, 128), row), pl.BlockSpec((bn, 1), row),
            pl.BlockSpec((128, 512), full), pl.BlockSpec((128, 512), full),
            pl.BlockSpec((16, 512), full), pl.BlockSpec((512, 128), full),
            pl.BlockSpec((1, 128), full),
        ],
        out_specs=[pl.BlockSpec((bn, 128), row),
                   pl.BlockSpec((16, 128), full),
                   pl.BlockSpec((16, 128), full)],
        out_shape=[jax.ShapeDtypeStruct((n, 128), F32),
                   jax.ShapeDtypeStruct((16, 128), F32),
                   jax.ShapeDtypeStruct((16, 128), F32)],
        compiler_params=pltpu.CompilerParams(
            dimension_semantics=("arbitrary",)),
        interpret=_INTERP,
    )(x, asum, cnt, bt, wn1x, wn1a, uun, wn2, bn2)


def _glob_mlp_call(uh, xm_sum, cb, em_sum, ce, wg1, bg1,
                   wg2, bg2, w1ue, b1e, w1un, b1n):
    def body(uh_r, xms_r, cb_r, ems_r, ce_r, w1_r, b1_r,
             w2_r, b2_r, wue_r, be_r, wun_r, bn_r, uh_o, uue_o, uun_o):
        xm = xms_r[...] / jnp.maximum(cb_r[...], 1.0)
        em = ems_r[...] / jnp.maximum(ce_r[...], 1.0)
        h = jnp.concatenate([uh_r[...], xm, em], axis=-1)
        pre = jnp.dot(h, w1_r[...], preferred_element_type=F32,
                      precision=_PHI) + b1_r[...]
        act = jnp.maximum(pre, 0.0)
        uhn = jnp.dot(act, w2_r[...], preferred_element_type=F32, precision=_PHI) + b2_r[...]
        uh_o[...] = uhn
        uue_o[...] = jnp.dot(uhn, wue_r[...], preferred_element_type=F32, precision=_PHI) + be_r[...]
        uun_o[...] = jnp.dot(uhn, wun_r[...], preferred_element_type=F32, precision=_PHI) + bn_r[...]

    z = lambda i: (0, 0)
    return pl.pallas_call(
        body,
        grid=(1,),
        in_specs=[
            pl.BlockSpec((16, 128), z), pl.BlockSpec((16, 128), z),
            pl.BlockSpec((16, 128), z), pl.BlockSpec((16, 128), z),
            pl.BlockSpec((16, 128), z),
            pl.BlockSpec((384, 512), z), pl.BlockSpec((1, 512), z),
            pl.BlockSpec((512, 128), z), pl.BlockSpec((1, 128), z),
            pl.BlockSpec((128, 512), z), pl.BlockSpec((1, 512), z),
            pl.BlockSpec((128, 512), z), pl.BlockSpec((1, 512), z),
        ],
        out_specs=[pl.BlockSpec((16, 128), z), pl.BlockSpec((16, 512), z),
                   pl.BlockSpec((16, 512), z)],
        out_shape=[jax.ShapeDtypeStruct((16, 128), F32),
                   jax.ShapeDtypeStruct((16, 512), F32),
                   jax.ShapeDtypeStruct((16, 512), F32)],
        interpret=_INTERP,
    )(uh, xm_sum, cb, em_sum, ce, wg1, bg1, wg2, bg2,
      w1ue, b1e, w1un, b1n)


def _head_call(xm_sum, cb, uh, wa, ba, wc1, bc1, wc2t, bc2):
    def body(xms_r, cb_r, uh_r, wa_r, ba_r, wc1_r, bc1_r, wc2_r, bc2_r,
             lg_o, val_o):
        xm = xms_r[...] / jnp.maximum(cb_r[...], 1.0)
        lg_o[...] = jnp.dot(xm, wa_r[...], preferred_element_type=F32, precision=_PHI) + ba_r[...]
        h = jnp.maximum(
            jnp.dot(uh_r[...], wc1_r[...], preferred_element_type=F32, precision=_PHI)
            + bc1_r[...], 0.0)
        v8 = jnp.dot(h, wc2_r[...], preferred_element_type=F32, precision=_PHI)
        val_o[...] = v8[:, 0:1] + bc2_r[...]

    z = lambda i: (0, 0)
    return pl.pallas_call(
        body,
        grid=(1,),
        in_specs=[
            pl.BlockSpec((16, 128), z), pl.BlockSpec((16, 128), z),
            pl.BlockSpec((16, 128), z),
            pl.BlockSpec((128, 8), z), pl.BlockSpec((1, 8), z),
            pl.BlockSpec((128, 128), z), pl.BlockSpec((1, 128), z),
            pl.BlockSpec((128, 8), z), pl.BlockSpec((1, 1), z),
        ],
        out_specs=[pl.BlockSpec((16, 8), z), pl.BlockSpec((16, 1), z)],
        out_shape=[jax.ShapeDtypeStruct((16, 8), F32),
                   jax.ShapeDtypeStruct((16, 1), F32)],
        interpret=_INTERP,
    )(xm_sum, cb, uh, wa, ba, wc1, bc1, wc2t, bc2)


def _r2(v):
    return v.reshape(1, -1)


def kernel(input, node_embedding, edge_attr, edge_index, u, batch, params):
    p = params
    n, d = input.shape
    e_n = edge_attr.shape[0]
    g = u.shape[0]
    h = 8
    bn = 1000
    be = 2048
    epad = 163840

    row = edge_index[0]
    col = edge_index[1]
    row_p = jnp.concatenate([row, jnp.zeros((epad - e_n,), jnp.int32)])
    col_p = jnp.concatenate([col, jnp.zeros((epad - e_n,), jnp.int32)])
    row2d = row_p.reshape(-1, 128)
    col2d = col_p.reshape(-1, 128)
    half = n // 2
    dump = half + (jnp.arange(epad, dtype=jnp.int32) & 7)
    in0 = col_p < half
    col_sc = jnp.stack([jnp.where(in0, col_p, dump),
                        jnp.where(in0, dump, col_p - half)]).reshape(2, -1, 128)
    ea_p = jnp.concatenate([edge_attr, jnp.zeros((epad - e_n, 16), F32)])
    batch2 = batch.reshape(-1, 1)
    zeros_n128 = jnp.zeros((n, 128), F32)

    lane = jnp.arange(128)
    sel = (lane[:, None] // 16 == jnp.arange(h)[None, :]).astype(F32)
    rep = sel.T

    wf = p["fusion"]["w"]
    wqkvs = jnp.concatenate(
        [p["q"]["w"], p["k"]["w"], p["v"]["w"], p["skip"]["w"]], axis=1)
    bqkvs = jnp.concatenate(
        [p["q"]["b"], p["k"]["b"], p["v"]["b"], p["skip"]["b"]]).reshape(1, -1)

    x0, q, k, v, skip = _node_prep_call(
        input, node_embedding,
        p["in_lin"]["w"], _r2(p["in_lin"]["b"]),
        p["emb_lin"]["w"], _r2(p["emb_lin"]["b"]),
        wf, _r2(p["fusion"]["b"]),
        _r2(p["ln1_g"]), _r2(p["ln1_b"]), wqkvs, bqkvs, bn)

    kg = _sc_gather_rows(k, row2d, 128, epad)
    vg = _sc_gather_rows(v, row2d, 128, epad)
    qg = _sc_gather_rows(q, col2d, 128, epad)
    br = _sc_gather_elems(batch, row2d, epad)

    contrib, dencnt_e, e0 = _attn_edge_call(
        kg, vg, qg, ea_p, p["e"]["w"], _r2(p["e"]["b"]),
        p["edge_enc"]["w"], _r2(p["edge_enc"]["b"]), sel, rep, e_n, be)

    msg = _sc_scatter_add(contrib, col_sc, n, zeros_n128)
    dencnt = _sc_scatter_add(dencnt_e, col_sc, n, zeros_n128)

    x = _attn_node_call(
        x0, skip, msg, dencnt, rep,
        _r2(p["ln2_g"]), _r2(p["ln2_b"]),
        p["ff1"]["w"], _r2(p["ff1"]["b"]), p["ff2"]["w"], _r2(p["ff2"]["b"]),
        _r2(p["fn_g"]), _r2(p["fn_b"]), bn)

    m0 = p["meta"][0]
    uh, uue, uun = _glob_prep_call(
        u, p["glob_enc"]["w"], _r2(p["glob_enc"]["b"]),
        m0["e1"]["w"][384:512], _r2(m0["e1"]["b"]),
        m0["n1"]["w"][256:384], _r2(m0["n1"]["b"]))

    e = e0
    br2 = br.reshape(-1, 1)
    xm_sum, cb = None, None
    for li in range(2):
        lp = p["meta"][li]
        xg = _sc_gather_rows(x, row2d, 128, epad)
        xcg = _sc_gather_rows(x, col2d, 128, epad)
        enew, em_sum, ce = _edge_mlp_call(
            xg, xcg, e, br2,
            lp["e1"]["w"][0:128], lp["e1"]["w"][128:256],
            lp["e1"]["w"][256:384], uue,
            lp["e2"]["w"], _r2(lp["e2"]["b"]), e_n, be)
        e = enew
        aggp = _sc_scatter_add(enew, col_sc, n, zeros_n128)
        x, xm_sum, cb = _node_mlp_call(
            x, aggp, dencnt, batch2,
            lp["n1"]["w"][0:128], lp["n1"]["w"][128:256], uun,
            lp["n2"]["w"], _r2(lp["n2"]["b"]), bn)
        if li + 1 < 2:
            nxt = p["meta"][li + 1]
            wue, be1 = nxt["e1"]["w"][384:512], _r2(nxt["e1"]["b"])
            wun, bn1 = nxt["n1"]["w"][256:384], _r2(nxt["n1"]["b"])
        else:
            wue, be1 = jnp.zeros((128, 512), F32), jnp.zeros((1, 512), F32)
            wun, bn1 = jnp.zeros((128, 512), F32), jnp.zeros((1, 512), F32)
        uh, uue, uun = _glob_mlp_call(
            uh, xm_sum, cb, em_sum, ce,
            lp["g1"]["w"], _r2(lp["g1"]["b"]),
            lp["g2"]["w"], _r2(lp["g2"]["b"]), wue, be1, wun, bn1)

    logits, value = _head_call(
        xm_sum, cb, uh, p["actor"]["w"], _r2(p["actor"]["b"]),
        p["c1"]["w"], _r2(p["c1"]["b"]),
        jnp.pad(p["c2"]["w"], ((0, 0), (0, 7))), _r2(p["c2"]["b"]))
    return logits, value


# speedup vs baseline: 6.1541x; 1.0161x over previous
import functools

import jax
import jax.numpy as jnp
from jax import lax
from jax.experimental import pallas as pl
from jax.experimental.pallas import tpu as pltpu
from jax.experimental.pallas import tpu_sc as plsc

F32 = jnp.float32
_PHI = jax.lax.Precision.DEFAULT
_PHX = jax.lax.Precision.HIGHEST
_INTERP = False


_NC, _NS, _L = 2, 16, 16
_NW = _NC * _NS
_GRP = 8
_SUB = 4


def _sc_mesh():
    return plsc.VectorSubcoreMesh(core_axis_name="c", subcore_axis_name="s")


def _sc_gather_rows(table, idx2d, width, out_rows, dtype=F32):
    R = idx2d.shape[0]
    per_w = R // _NW
    n_steps = per_w // _GRP
    cs = _SUB * 128

    @functools.partial(
        pl.kernel,
        out_type=jax.ShapeDtypeStruct((out_rows, width), dtype),
        mesh=_sc_mesh(),
        scratch_types=[
            pltpu.VMEM((_GRP, 128), jnp.int32),
            pltpu.VMEM((cs, width), dtype),
        ] + [pltpu.SemaphoreType.DMA] * 8,
    )
    def k(table_hbm, idx_hbm, out_hbm, idx_v, dat_v, *sems):
        gsem, osem = sems[0:4], sems[4:8]
        w = lax.axis_index("c") * _NS + lax.axis_index("s")
        row0 = w * per_w

        def quarter(q):
            return dat_v.at[pl.ds(q * 128, 128)]

        def body(t, _):
            r0 = row0 + t * _GRP
            pltpu.sync_copy(idx_hbm.at[pl.ds(r0, _GRP)], idx_v)
            g1 = [pltpu.async_copy(table_hbm.at[idx_v.at[j]],
                                   quarter(j), gsem[j]) for j in range(4)]
            o1 = []
            for j in range(4):
                g1[j].wait()
                o1.append(pltpu.async_copy(
                    quarter(j), out_hbm.at[pl.ds((r0 + j) * 128, 128)],
                    osem[j]))
            g2 = []
            for j in range(4):
                o1[j].wait()
                g2.append(pltpu.async_copy(table_hbm.at[idx_v.at[4 + j]],
                                           quarter(j), gsem[j]))
            o2 = []
            for j in range(4):
                g2[j].wait()
                o2.append(pltpu.async_copy(
                    quarter(j), out_hbm.at[pl.ds((r0 + 4 + j) * 128, 128)],
                    osem[j]))
            for j in range(4):
                o2[j].wait()
            return 0

        lax.fori_loop(0, n_steps, body, 0)

    return k(table, idx2d)


def _sc_gather_elems(table, idx2d, out_rows):
    R = idx2d.shape[0]
    per_w = R // _NW
    n_steps = per_w // _GRP
    cs = _GRP * 128

    @functools.partial(
        pl.kernel,
        out_type=jax.ShapeDtypeStruct((out_rows,), jnp.int32),
        mesh=_sc_mesh(),
        scratch_types=[
            pltpu.VMEM((_GRP, 128), jnp.int32),
            pltpu.VMEM((cs,), jnp.int32),
            pltpu.SemaphoreType.DMA,
        ],
    )
    def k(table_hbm, idx_hbm, out_hbm, idx_v, dat_v, sem):
        w = lax.axis_index("c") * _NS + lax.axis_index("s")
        row0 = w * per_w

        def body(t, _):
            r0 = row0 + t * _GRP
            pltpu.sync_copy(idx_hbm.at[pl.ds(r0, _GRP)], idx_v)
            cps = [pltpu.async_copy(
                table_hbm.at[idx_v.at[j]],
                dat_v.at[pl.ds(j * 128, 128)], sem) for j in range(_GRP)]
            for cp in cps:
                cp.wait()
            pltpu.sync_copy(dat_v, out_hbm.at[pl.ds(r0 * 128, cs)])
            return 0

        lax.fori_loop(0, n_steps, body, 0)

    return k(table, idx2d)


def _sc_scatter_add(data, idx3d, n_rows, zeros_tab):
    epad, width = data.shape
    R = idx3d.shape[1]
    half_n = n_rows // 2
    tab_rows = half_n + 8
    per_s = R // _NS
    n_steps = per_s // _GRP
    cs = 2 * 128
    zmain, ztail = (tab_rows // _NS) // 8 * 8, None
    ztail = tab_rows - zmain * _NS
    emain = (half_n // _NS) // 8 * 8
    etail = half_n - emain * _NS

    @functools.partial(
        pl.kernel,
        out_type=jax.ShapeDtypeStruct((n_rows, width), F32),
        mesh=_sc_mesh(),
        scratch_types=[
            pltpu.VMEM((_GRP, 128), jnp.int32),
            pltpu.VMEM((cs, width), F32),
            pltpu.VMEM((cs, width), F32),
            pltpu.VMEM_SHARED((tab_rows, width), F32),
            pltpu.SemaphoreType.DMA,
            pltpu.SemaphoreType.DMA,
            pltpu.SemaphoreType.DMA,
        ],
    )
    def k(data_hbm, idx_hbm, zeros_hbm, out_hbm, idx_v, dat_va, dat_vb,
          shared, sem_a, sem_b, lsem):
        c = lax.axis_index("c")
        s = lax.axis_index("s")
        pltpu.sync_copy(zeros_hbm.at[pl.ds(s * zmain, zmain)],
                        shared.at[pl.ds(s * zmain, zmain)])
        if ztail:
            @pl.when(s == _NS - 1)
            def _():
                pltpu.sync_copy(zeros_hbm.at[pl.ds(_NS * zmain, ztail)],
                                shared.at[pl.ds(_NS * zmain, ztail)])
        plsc.subcore_barrier()
        row0 = s * per_s

        def body(t, _):
            r0 = row0 + t * _GRP
            pltpu.sync_copy(idx_hbm.at[c, pl.ds(r0, _GRP)], idx_v)
            for h in range(2):
                base = r0 + h * 4
                la = pltpu.async_copy(
                    data_hbm.at[pl.ds(base * 128, cs)], dat_va, lsem)
                lb = pltpu.async_copy(
                    data_hbm.at[pl.ds((base + 2) * 128, cs)], dat_vb, lsem)
                la.wait()
                ca = [pltpu.async_copy(
                    dat_va.at[pl.ds(j * 128, 128)],
                    shared.at[idx_v.at[h * 4 + j]], sem_a, add=True)
                    for j in range(2)]
                lb.wait()
                cb = [pltpu.async_copy(
                    dat_vb.at[pl.ds(j * 128, 128)],
                    shared.at[idx_v.at[h * 4 + 2 + j]], sem_b, add=True)
                    for j in range(2)]
                for cp in ca + cb:
                    cp.wait()
            return 0

        lax.fori_loop(0, n_steps, body, 0)
        plsc.subcore_barrier()
        pltpu.sync_copy(shared.at[pl.ds(s * emain, emain)],
                        out_hbm.at[pl.ds(c * half_n + s * emain, emain)])
        if etail:
            @pl.when(s == _NS - 1)
            def _():
                pltpu.sync_copy(
                    shared.at[pl.ds(_NS * emain, etail)],
                    out_hbm.at[pl.ds(c * half_n + _NS * emain, etail)])

    return k(data, idx3d, zeros_tab)


def _ln(x, g, b):
    m = jnp.mean(x, axis=-1, keepdims=True)
    d = x - m
    v = jnp.mean(d * d, axis=-1, keepdims=True)
    return d / jnp.sqrt(v + 1e-5) * g + b


def _node_prep_call(inp, emb, wi, bi, we, be, wf, bf, g1, b1,
                    wqkvs, bqkvs, bn):
    n = inp.shape[0]
    grid = (n // bn,)

    def body(inp_r, emb_r, wi_r, bi_r, we_r, be_r, wf_r, bf_r,
             g1_r, b1_r, wq_r, bq_r, x0_o, q_o, k_o, v_o, s_o):
        a = jnp.dot(inp_r[...], wi_r[...], preferred_element_type=F32, precision=_PHI) + bi_r[...]
        ne = jnp.dot(emb_r[...], we_r[...], preferred_element_type=F32, precision=_PHI) + be_r[...]
        comb = jnp.concatenate([ne, a, ne], axis=-1)
        x0 = jnp.dot(comb, wf_r[...], preferred_element_type=F32,
                     precision=_PHI) + bf_r[...]
        xn = _ln(x0, g1_r[...], b1_r[...])
        qkvs = jnp.dot(xn, wq_r[...], preferred_element_type=F32, precision=_PHI) + bq_r[...]
        x0_o[...] = x0
        q_o[...] = qkvs[:, 0:128]
        k_o[...] = qkvs[:, 128:256]
        v_o[...] = qkvs[:, 256:384]
        s_o[...] = qkvs[:, 384:512]

    row = lambda i: (i, 0)
    full = lambda i: (0, 0)
    oshape = jax.ShapeDtypeStruct((n, 128), F32)
    return pl.pallas_call(
        body,
        grid=grid,
        in_specs=[
            pl.BlockSpec((bn, 128), row), pl.BlockSpec((bn, 128), row),
            pl.BlockSpec((128, 128), full), pl.BlockSpec((1, 128), full),
            pl.BlockSpec((128, 128), full), pl.BlockSpec((1, 128), full),
            pl.BlockSpec((384, 128), full),
            pl.BlockSpec((1, 128), full), pl.BlockSpec((1, 128), full),
            pl.BlockSpec((1, 128), full),
            pl.BlockSpec((128, 512), full), pl.BlockSpec((1, 512), full),
        ],
        out_specs=[pl.BlockSpec((bn, 128), row)] * 5,
        out_shape=[oshape] * 5,
        interpret=_INTERP,
    )(inp, emb, wi, bi, we, be, wf, bf, g1, b1, wqkvs, bqkvs)


def _attn_edge_call(kg, vg, qg, ea, watt, batt, wenc, benc, sel, rep,
                    n_edges, be):
    epad = kg.shape[0]
    grid = (epad // be,)

    def body(kg_r, vg_r, qg_r, ea_r, watt_r, batt_r, wenc_r, benc_r,
             sel_r, rep_r, contrib_o, dencnt_o, e0_o):
        i = pl.program_id(0)
        rows = lax.broadcasted_iota(jnp.int32, (be, 1), 0) + i * be
        mask = (rows < n_edges).astype(F32)
        ep = jnp.dot(ea_r[...], watt_r[...], preferred_element_type=F32, precision=_PHI) + batt_r[...]
        ke = kg_r[...] + ep
        ve = vg_r[...] + ep
        alpha = jnp.dot(qg_r[...] * ke, sel_r[...],
                        preferred_element_type=F32, precision=_PHX) * 0.25
        ex = jnp.exp(alpha) * mask
        contrib_o[...] = ve * jnp.dot(ex, rep_r[...],
                                      preferred_element_type=F32,
                                      precision=_PHX)
        dencnt_o[...] = jnp.concatenate(
            [ex, jnp.broadcast_to(mask, (be, 8)), jnp.zeros((be, 112), F32)],
            axis=-1)
        e0_o[...] = (jnp.dot(ea_r[...], wenc_r[...],
                             preferred_element_type=F32, precision=_PHI) + benc_r[...]) * mask

    row = lambda i: (i, 0)
    full = lambda i: (0, 0)
    return pl.pallas_call(
        body,
        grid=grid,
        in_specs=[
            pl.BlockSpec((be, 128), row), pl.BlockSpec((be, 128), row),
            pl.BlockSpec((be, 128), row), pl.BlockSpec((be, 16), row),
            pl.BlockSpec((16, 128), full), pl.BlockSpec((1, 128), full),
            pl.BlockSpec((16, 128), full), pl.BlockSpec((1, 128), full),
            pl.BlockSpec((128, 8), full), pl.BlockSpec((8, 128), full),
        ],
        out_specs=[pl.BlockSpec((be, 128), row), pl.BlockSpec((be, 128), row),
                   pl.BlockSpec((be, 128), row)],
        out_shape=[jax.ShapeDtypeStruct((epad, 128), F32),
                   jax.ShapeDtypeStruct((epad, 128), F32),
                   jax.ShapeDtypeStruct((epad, 128), F32)],
        interpret=_INTERP,
    )(kg, vg, qg, ea, watt, batt, wenc, benc, sel, rep)


def _attn_node_call(x0, skip, msg, dencnt, rep, g2, b2, wf1, bf1,
                    wf2, bf2, fng, fnb, bn):
    n = x0.shape[0]
    grid = (n // bn,)

    def body(x0_r, sk_r, m_r, d_r, rep_r, g2_r, b2_r,
             wf1_r, bf1_r, wf2_r, bf2_r, fng_r, fnb_r, x_o):
        den = jnp.dot(d_r[:, 0:8], rep_r[...],
                      preferred_element_type=F32, precision=_PHX) + 1e-16
        x1 = x0_r[...] + m_r[...] / den + sk_r[...]
        xn = _ln(x1, g2_r[...], b2_r[...])
        h = jnp.maximum(
            jnp.dot(xn, wf1_r[...], preferred_element_type=F32, precision=_PHI) + bf1_r[...], 0.0)
        ff = jnp.dot(h, wf2_r[...], preferred_element_type=F32, precision=_PHI) + bf2_r[...]
        x_o[...] = _ln(x1 + ff, fng_r[...], fnb_r[...])

    row = lambda i: (i, 0)
    full = lambda i: (0, 0)
    return pl.pallas_call(
        body,
        grid=grid,
        in_specs=[
            pl.BlockSpec((bn, 128), row), pl.BlockSpec((bn, 128), row),
            pl.BlockSpec((bn, 128), row), pl.BlockSpec((bn, 128), row),
            pl.BlockSpec((8, 128), full),
            pl.BlockSpec((1, 128), full), pl.BlockSpec((1, 128), full),
            pl.BlockSpec((128, 512), full), pl.BlockSpec((1, 512), full),
            pl.BlockSpec((512, 128), full), pl.BlockSpec((1, 128), full),
            pl.BlockSpec((1, 128), full), pl.BlockSpec((1, 128), full),
        ],
        out_specs=[pl.BlockSpec((bn, 128), row)],
        out_shape=[jax.ShapeDtypeStruct((n, 128), F32)],
        interpret=_INTERP,
    )(x0, skip, msg, dencnt, rep, g2, b2, wf1, bf1, wf2, bf2, fng, fnb)[0]


def _glob_prep_call(u, wg, bg, w1ue, b1e, w1un, b1n):
    def body(u_r, wg_r, bg_r, wue_r, be_r, wun_r, bn_r, uh_o, uue_o, uun_o):
        uh = jnp.dot(u_r[...], wg_r[...], preferred_element_type=F32,
                     precision=_PHI) + bg_r[...]
        uh_o[...] = uh
        uue_o[...] = jnp.dot(uh, wue_r[...], preferred_element_type=F32, precision=_PHI) + be_r[...]
        uun_o[...] = jnp.dot(uh, wun_r[...], preferred_element_type=F32, precision=_PHI) + bn_r[...]

    full = lambda: (0, 0)
    return pl.pallas_call(
        body,
        grid=(1,),
        in_specs=[
            pl.BlockSpec((16, 16), lambda i: (0, 0)),
            pl.BlockSpec((16, 128), lambda i: (0, 0)),
            pl.BlockSpec((1, 128), lambda i: (0, 0)),
            pl.BlockSpec((128, 512), lambda i: (0, 0)),
            pl.BlockSpec((1, 512), lambda i: (0, 0)),
            pl.BlockSpec((128, 512), lambda i: (0, 0)),
            pl.BlockSpec((1, 512), lambda i: (0, 0)),
        ],
        out_specs=[pl.BlockSpec((16, 128), lambda i: (0, 0)),
                   pl.BlockSpec((16, 512), lambda i: (0, 0)),
                   pl.BlockSpec((16, 512), lambda i: (0, 0))],
        out_shape=[jax.ShapeDtypeStruct((16, 128), F32),
                   jax.ShapeDtypeStruct((16, 512), F32),
                   jax.ShapeDtypeStruct((16, 512), F32)],
        interpret=_INTERP,
    )(u, wg, bg, w1ue, b1e, w1un, b1n)


def _edge_mlp_call(xg, xcg, e, br, w1r, w1c, w1e, uue, w2, b2, n_edges, be):
    epad = xg.shape[0]
    grid = (epad // be,)

    def body(xg_r, xcg_r, e_r, br_r, w1r_r, w1c_r, w1e_r, uue_r, w2_r, b2_r,
             enew_o, em_o, ce_o):
        i = pl.program_id(0)
        rows = lax.broadcasted_iota(jnp.int32, (be, 1), 0) + i * be
        mask = (rows < n_edges).astype(F32)
        oh = (br_r[...] == lax.broadcasted_iota(jnp.int32, (be, 16), 1)
              ).astype(F32) * mask
        pre = (jnp.dot(xg_r[...], w1r_r[...],
                       preferred_element_type=F32, precision=_PHI)
               + jnp.dot(xcg_r[...], w1c_r[...],
                         preferred_element_type=F32, precision=_PHI)
               + jnp.dot(e_r[...], w1e_r[...], preferred_element_type=F32, precision=_PHI)
               + jnp.dot(oh, uue_r[...], preferred_element_type=F32,
                         precision=_PHX))
        act = jnp.maximum(pre, 0.0)
        en = (jnp.dot(act, w2_r[...], preferred_element_type=F32, precision=_PHI)
              + b2_r[...]) * mask
        enew_o[...] = en

        @pl.when(i == 0)
        def _():
            em_o[...] = jnp.zeros_like(em_o)
            ce_o[...] = jnp.zeros_like(ce_o)

        em_o[...] += lax.dot_general(oh, en, (((0,), (0,)), ((), ())),
                                     preferred_element_type=F32, precision=_PHX)
        ce_o[...] += lax.dot_general(oh, jnp.ones((be, 128), F32),
                                     (((0,), (0,)), ((), ())),
                                     preferred_element_type=F32, precision=_PHX)

    row = lambda i: (i, 0)
    full = lambda i: (0, 0)
    return pl.pallas_call(
        body,
        grid=grid,
        in_specs=[
            pl.BlockSpec((be, 128), row), pl.BlockSpec((be, 128), row),
            pl.BlockSpec((be, 128), row), pl.BlockSpec((be, 1), row),
            pl.BlockSpec((128, 512), full), pl.BlockSpec((128, 512), full),
            pl.BlockSpec((128, 512), full), pl.BlockSpec((16, 512), full),
            pl.BlockSpec((512, 128), full), pl.BlockSpec((1, 128), full),
        ],
        out_specs=[pl.BlockSpec((be, 128), row),
                   pl.BlockSpec((16, 128), full),
                   pl.BlockSpec((16, 128), full)],
        out_shape=[jax.ShapeDtypeStruct((epad, 128), F32),
                   jax.ShapeDtypeStruct((16, 128), F32),
                   jax.ShapeDtypeStruct((16, 128), F32)],
        compiler_params=pltpu.CompilerParams(
            dimension_semantics=("arbitrary",)),
        interpret=_INTERP,
    )(xg, xcg, e, br, w1r, w1c, w1e, uue, w2, b2)


def _node_mlp_call(x, asum, cnt, bt, wn1x, wn1a, uun, wn2, bn2, bn):
    n = x.shape[0]
    grid = (n // bn,)

    def body(x_r, a_r, cnt_r, bt_r, w1x_r, w1a_r, uun_r, w2_r, b2_r,
             xn_o, xm_o, cb_o):
        inv = 1.0 / jnp.maximum(cnt_r[:, 8:9], 1.0)
        agg = a_r[...] * inv
        oh = (bt_r[...] == lax.broadcasted_iota(jnp.int32, (bn, 16), 1)
              ).astype(F32)
        pre = (jnp.dot(x_r[...], w1x_r[...], preferred_element_type=F32, precision=_PHI)
               + jnp.dot(agg, w1a_r[...], preferred_element_type=F32, precision=_PHI)
               + jnp.dot(oh, uun_r[...], preferred_element_type=F32,
                         precision=_PHX))
        act = jnp.maximum(pre, 0.0)
        xn = jnp.dot(act, w2_r[...], preferred_element_type=F32, precision=_PHI) + b2_r[...]
        xn_o[...] = xn

        @pl.when(pl.program_id(0) == 0)
        def _():
            xm_o[...] = jnp.zeros_like(xm_o)
            cb_o[...] = jnp.zeros_like(cb_o)

        xm_o[...] += lax.dot_general(oh, xn, (((0,), (0,)), ((), ())),
                                     preferred_element_type=F32, precision=_PHX)
        cb_o[...] += lax.dot_general(oh, jnp.ones((bn, 128), F32),
                                     (((0,), (0,)), ((), ())),
                                     preferred_element_type=F32, precision=_PHX)

    row = lambda i: (i, 0)
    full = lambda i: (0, 0)
    return pl.pallas_call(
        body,
        grid=grid,
        in_specs=[
            pl.BlockSpec((bn, 128), row), pl.BlockSpec((bn, 128), row),
            pl.BlockSpec((bn, 128), row), pl.BlockSpec((bn, 1), row),
            pl.BlockSpec((128, 512), full), pl.BlockSpec((128, 512), full),
            pl.BlockSpec((16, 512), full), pl.BlockSpec((512, 128), full),
            pl.BlockSpec((1, 128), full),
        ],
        out_specs=[pl.BlockSpec((bn, 128), row),
                   pl.BlockSpec((16, 128), full),
                   pl.BlockSpec((16, 128), full)],
        out_shape=[jax.ShapeDtypeStruct((n, 128), F32),
                   jax.ShapeDtypeStruct((16, 128), F32),
                   jax.ShapeDtypeStruct((16, 128), F32)],
        compiler_params=pltpu.CompilerParams(
            dimension_semantics=("arbitrary",)),
        interpret=_INTERP,
    )(x, asum, cnt, bt, wn1x, wn1a, uun, wn2, bn2)


def _glob_mlp_call(uh, xm_sum, cb, em_sum, ce, wg1, bg1,
                   wg2, bg2, w1ue, b1e, w1un, b1n):
    def body(uh_r, xms_r, cb_r, ems_r, ce_r, w1_r, b1_r,
             w2_r, b2_r, wue_r, be_r, wun_r, bn_r, uh_o, uue_o, uun_o):
        xm = xms_r[...] / jnp.maximum(cb_r[...], 1.0)
        em = ems_r[...] / jnp.maximum(ce_r[...], 1.0)
        h = jnp.concatenate([uh_r[...], xm, em], axis=-1)
        pre = jnp.dot(h, w1_r[...], preferred_element_type=F32,
                      precision=_PHI) + b1_r[...]
        act = jnp.maximum(pre, 0.0)
        uhn = jnp.dot(act, w2_r[...], preferred_element_type=F32, precision=_PHI) + b2_r[...]
        uh_o[...] = uhn
        uue_o[...] = jnp.dot(uhn, wue_r[...], preferred_element_type=F32, precision=_PHI) + be_r[...]
        uun_o[...] = jnp.dot(uhn, wun_r[...], preferred_element_type=F32, precision=_PHI) + bn_r[...]

    z = lambda i: (0, 0)
    return pl.pallas_call(
        body,
        grid=(1,),
        in_specs=[
            pl.BlockSpec((16, 128), z), pl.BlockSpec((16, 128), z),
            pl.BlockSpec((16, 128), z), pl.BlockSpec((16, 128), z),
            pl.BlockSpec((16, 128), z),
            pl.BlockSpec((384, 512), z), pl.BlockSpec((1, 512), z),
            pl.BlockSpec((512, 128), z), pl.BlockSpec((1, 128), z),
            pl.BlockSpec((128, 512), z), pl.BlockSpec((1, 512), z),
            pl.BlockSpec((128, 512), z), pl.BlockSpec((1, 512), z),
        ],
        out_specs=[pl.BlockSpec((16, 128), z), pl.BlockSpec((16, 512), z),
                   pl.BlockSpec((16, 512), z)],
        out_shape=[jax.ShapeDtypeStruct((16, 128), F32),
                   jax.ShapeDtypeStruct((16, 512), F32),
                   jax.ShapeDtypeStruct((16, 512), F32)],
        interpret=_INTERP,
    )(uh, xm_sum, cb, em_sum, ce, wg1, bg1, wg2, bg2,
      w1ue, b1e, w1un, b1n)


def _head_call(xm_sum, cb, uh, wa, ba, wc1, bc1, wc2t, bc2):
    def body(xms_r, cb_r, uh_r, wa_r, ba_r, wc1_r, bc1_r, wc2_r, bc2_r,
             lg_o, val_o):
        xm = xms_r[...] / jnp.maximum(cb_r[...], 1.0)
        lg_o[...] = jnp.dot(xm, wa_r[...], preferred_element_type=F32, precision=_PHI) + ba_r[...]
        h = jnp.maximum(
            jnp.dot(uh_r[...], wc1_r[...], preferred_element_type=F32, precision=_PHI)
            + bc1_r[...], 0.0)
        v8 = jnp.dot(h, wc2_r[...], preferred_element_type=F32, precision=_PHI)
        val_o[...] = v8[:, 0:1] + bc2_r[...]

    z = lambda i: (0, 0)
    return pl.pallas_call(
        body,
        grid=(1,),
        in_specs=[
            pl.BlockSpec((16, 128), z), pl.BlockSpec((16, 128), z),
            pl.BlockSpec((16, 128), z),
            pl.BlockSpec((128, 8), z), pl.BlockSpec((1, 8), z),
            pl.BlockSpec((128, 128), z), pl.BlockSpec((1, 128), z),
            pl.BlockSpec((128, 8), z), pl.BlockSpec((1, 1), z),
        ],
        out_specs=[pl.BlockSpec((16, 8), z), pl.BlockSpec((16, 1), z)],
        out_shape=[jax.ShapeDtypeStruct((16, 8), F32),
                   jax.ShapeDtypeStruct((16, 1), F32)],
        interpret=_INTERP,
    )(xm_sum, cb, uh, wa, ba, wc1, bc1, wc2t, bc2)


def _r2(v):
    return v.reshape(1, -1)


def kernel(input, node_embedding, edge_attr, edge_index, u, batch, params):
    p = params
    n, d = input.shape
    e_n = edge_attr.shape[0]
    g = u.shape[0]
    h = 8
    bn = 1000
    be = 2048
    epad = 163840

    row = edge_index[0]
    col = edge_index[1]
    row_p = jnp.concatenate([row, jnp.zeros((epad - e_n,), jnp.int32)])
    col_p = jnp.concatenate([col, jnp.zeros((epad - e_n,), jnp.int32)])
    row2d = row_p.reshape(-1, 128)
    col2d = col_p.reshape(-1, 128)
    half = n // 2
    dump = half + (jnp.arange(epad, dtype=jnp.int32) & 7)
    in0 = col_p < half
    col_sc = jnp.stack([jnp.where(in0, col_p, dump),
                        jnp.where(in0, dump, col_p - half)]).reshape(2, -1, 128)
    ea_p = jnp.concatenate([edge_attr, jnp.zeros((epad - e_n, 16), F32)])
    batch2 = batch.reshape(-1, 1)
    zeros_n128 = jnp.zeros((n, 128), F32)

    lane = jnp.arange(128)
    sel = (lane[:, None] // 16 == jnp.arange(h)[None, :]).astype(F32)
    rep = sel.T

    wf = p["fusion"]["w"]
    wqkvs = jnp.concatenate(
        [p["q"]["w"], p["k"]["w"], p["v"]["w"], p["skip"]["w"]], axis=1)
    bqkvs = jnp.concatenate(
        [p["q"]["b"], p["k"]["b"], p["v"]["b"], p["skip"]["b"]]).reshape(1, -1)

    x0, q, k, v, skip = _node_prep_call(
        input, node_embedding,
        p["in_lin"]["w"], _r2(p["in_lin"]["b"]),
        p["emb_lin"]["w"], _r2(p["emb_lin"]["b"]),
        wf, _r2(p["fusion"]["b"]),
        _r2(p["ln1_g"]), _r2(p["ln1_b"]), wqkvs, bqkvs, bn)

    kg = _sc_gather_rows(k, row2d, 128, epad)
    vg = _sc_gather_rows(v, row2d, 128, epad)
    qg = _sc_gather_rows(q, col2d, 128, epad)
    br = _sc_gather_elems(batch, row2d, epad)

    contrib, dencnt_e, e0 = _attn_edge_call(
        kg, vg, qg, ea_p, p["e"]["w"], _r2(p["e"]["b"]),
        p["edge_enc"]["w"], _r2(p["edge_enc"]["b"]), sel, rep, e_n, be)

    msg = _sc_scatter_add(contrib, col_sc, n, zeros_n128)
    dencnt = _sc_scatter_add(dencnt_e, col_sc, n, zeros_n128)

    x = _attn_node_call(
        x0, skip, msg, dencnt, rep,
        _r2(p["ln2_g"]), _r2(p["ln2_b"]),
        p["ff1"]["w"], _r2(p["ff1"]["b"]), p["ff2"]["w"], _r2(p["ff2"]["b"]),
        _r2(p["fn_g"]), _r2(p["fn_b"]), bn)

    m0 = p["meta"][0]
    uh, uue, uun = _glob_prep_call(
        u, p["glob_enc"]["w"], _r2(p["glob_enc"]["b"]),
        m0["e1"]["w"][384:512], _r2(m0["e1"]["b"]),
        m0["n1"]["w"][256:384], _r2(m0["n1"]["b"]))

    e = e0
    br2 = br.reshape(-1, 1)
    xm_sum, cb = None, None
    for li in range(2):
        lp = p["meta"][li]
        xg = _sc_gather_rows(x, row2d, 128, epad)
        xcg = _sc_gather_rows(x, col2d, 128, epad)
        enew, em_sum, ce = _edge_mlp_call(
            xg, xcg, e, br2,
            lp["e1"]["w"][0:128], lp["e1"]["w"][128:256],
            lp["e1"]["w"][256:384], uue,
            lp["e2"]["w"], _r2(lp["e2"]["b"]), e_n, be)
        e = enew
        aggp = _sc_scatter_add(enew, col_sc, n, zeros_n128)
        x, xm_sum, cb = _node_mlp_call(
            x, aggp, dencnt, batch2,
            lp["n1"]["w"][0:128], lp["n1"]["w"][128:256], uun,
            lp["n2"]["w"], _r2(lp["n2"]["b"]), bn)
        if li + 1 < 2:
            nxt = p["meta"][li + 1]
            wue, be1 = nxt["e1"]["w"][384:512], _r2(nxt["e1"]["b"])
            wun, bn1 = nxt["n1"]["w"][256:384], _r2(nxt["n1"]["b"])
        else:
            wue, be1 = jnp.zeros((128, 512), F32), jnp.zeros((1, 512), F32)
            wun, bn1 = jnp.zeros((128, 512), F32), jnp.zeros((1, 512), F32)
        uh, uue, uun = _glob_mlp_call(
            uh, xm_sum, cb, em_sum, ce,
            lp["g1"]["w"], _r2(lp["g1"]["b"]),
            lp["g2"]["w"], _r2(lp["g2"]["b"]), wue, be1, wun, bn1)

    logits, value = _head_call(
        xm_sum, cb, uh, p["actor"]["w"], _r2(p["actor"]["b"]),
        p["c1"]["w"], _r2(p["c1"]["b"]),
        jnp.pad(p["c2"]["w"], ((0, 0), (0, 7))), _r2(p["c2"]["b"]))
    return logits, value
